# Initial kernel scaffold; baseline (speedup 1.0000x reference)
import jax
import jax.numpy as jnp
from jax.experimental import pallas as pl


def kernel(x):
    raise NotImplementedError("write your pallas kernel here")


# speedup vs baseline: 1.7059x; 1.7059x over previous
import jax
import jax.numpy as jnp
from jax.experimental import pallas as pl

_TOP_K = 64
_TOP_P = 0.6
_ROWS_PER_STEP = 8
_NEG_BIG = -1.0


def _body(x_ref, o_ref):
    x = x_ref[...]
    r, n = x.shape
    m = jnp.max(x, axis=-1, keepdims=True)
    e = jnp.exp(x - m)
    s = e / jnp.sum(e, axis=-1, keepdims=True)

    lane = jax.lax.broadcasted_iota(jnp.int32, (r, n), 1)
    k_iota = jax.lax.broadcasted_iota(jnp.int32, (r, _TOP_K), 1)

    def extract(j, carry):
        work, vals = carry
        mx = jnp.max(work, axis=-1, keepdims=True)
        idx = jnp.min(jnp.where(work == mx, lane, n), axis=-1, keepdims=True)
        work = jnp.where(lane == idx, _NEG_BIG, work)
        vals = jnp.where(k_iota == j, mx, vals)
        return work, vals

    vals = jnp.zeros((r, _TOP_K), jnp.float32)
    _, vals = jax.lax.fori_loop(0, _TOP_K, extract, (s, vals))

    tri_u = jax.lax.broadcasted_iota(jnp.int32, (_TOP_K, _TOP_K), 0)
    tri_j = jax.lax.broadcasted_iota(jnp.int32, (_TOP_K, _TOP_K), 1)
    tri = (tri_u <= tri_j).astype(jnp.float32)
    cum = jnp.dot(vals, tri, preferred_element_type=jnp.float32)
    over = cum > _TOP_P
    first_over = jnp.min(jnp.where(over, k_iota, _TOP_K), axis=-1)
    kk = jnp.minimum(first_over + 1, _TOP_K)

    s64 = s[:, :_TOP_K]
    gt = jnp.zeros((r, _TOP_K), jnp.int32)
    for v in range(_TOP_K):
        t_v = s[:, v][:, None]
        cnt = jnp.sum((s > t_v).astype(jnp.int32), axis=-1, keepdims=True)
        gt = jnp.where(k_iota == v, cnt, gt)
    a = s64[:, :, None]
    b = s64[:, None, :]
    iu = jax.lax.broadcasted_iota(jnp.int32, (_TOP_K, _TOP_K), 1)[None]
    iv = jax.lax.broadcasted_iota(jnp.int32, (_TOP_K, _TOP_K), 0)[None]
    eq = jnp.sum(((a == b) & (iu < iv)).astype(jnp.int32), axis=-1)
    rank = gt + eq

    out = jnp.full((r, n), -jnp.inf, jnp.float32)
    for v in range(_TOP_K):
        rv = rank[:, v][:, None]
        keep = (kk > v)[:, None]
        out = jnp.where((lane == rv) & keep, vals[:, v][:, None], out)
    o_ref[...] = out


@jax.jit
def kernel(x):
    rows, n = x.shape
    grid = rows // _ROWS_PER_STEP
    return pl.pallas_call(
        _body,
        grid=(grid,),
        in_specs=[pl.BlockSpec((_ROWS_PER_STEP, n), lambda i: (i, 0))],
        out_specs=pl.BlockSpec((_ROWS_PER_STEP, n), lambda i: (i, 0)),
        out_shape=jax.ShapeDtypeStruct((rows, n), jnp.float32),
    )(x)


# speedup vs baseline: 3.9311x; 2.3044x over previous
import functools

import jax
import jax.numpy as jnp
from jax import lax
from jax.experimental import pallas as pl
from jax.experimental.pallas import tpu as pltpu
from jax.experimental.pallas import tpu_sc as plsc

_TOP_K = 64
_TOP_P = 0.6
_N = 32768
_ROWS = 128
_L = 16
_NV = _N // _L
_NW = 32
_ROWS_PER_W = _ROWS // _NW
_NEG = -1.0

_IOTA = None


def _vmax(v):
    return jnp.max(v)


def _ffs(mask):
    return jnp.max(plsc.all_reduce_ffs(mask))


def _cminmax(ak, av, bk, bv):
    c = ak <= bk
    lo_k = jnp.where(c, ak, bk)
    lo_v = jnp.where(c, av, bv)
    hi_k = jnp.where(c, bk, ak)
    hi_v = jnp.where(c, bv, av)
    return lo_k, lo_v, hi_k, hi_v


def _merge16(ak, av, bk, bv):
    rbk = lax.rev(bk, (0,))
    rbv = lax.rev(bv, (0,))
    lo_k, lo_v, hi_k, hi_v = _cminmax(ak, av, rbk, rbv)
    lo_k, lo_v = plsc.sort_key_val(lo_k, lo_v)
    hi_k, hi_v = plsc.sort_key_val(hi_k, hi_v)
    return (lo_k, hi_k), (lo_v, hi_v)


def _merge32(aks, avs, bks, bvs):
    rbk = (lax.rev(bks[1], (0,)), lax.rev(bks[0], (0,)))
    rbv = (lax.rev(bvs[1], (0,)), lax.rev(bvs[0], (0,)))
    l0k, l0v, h0k, h0v = _cminmax(aks[0], avs[0], rbk[0], rbv[0])
    l1k, l1v, h1k, h1v = _cminmax(aks[1], avs[1], rbk[1], rbv[1])
    p0k, p0v, p1k, p1v = _cminmax(l0k, l0v, l1k, l1v)
    q0k, q0v, q1k, q1v = _cminmax(h0k, h0v, h1k, h1v)
    p0k, p0v = plsc.sort_key_val(p0k, p0v)
    p1k, p1v = plsc.sort_key_val(p1k, p1v)
    q0k, q0v = plsc.sort_key_val(q0k, q0v)
    q1k, q1v = plsc.sort_key_val(q1k, q1v)
    return (p0k, p1k, q0k, q1k), (p0v, p1v, q0v, q1v)


def _sc_body(x_hbm, o_hbm, s_v, o_v, m1_v, m2_v, tkey_v, tperm_v, rank_v,
             hist_v, sem):
    iota = lax.broadcasted_iota(jnp.int32, (_L,), 0)
    neg_inf_v = jnp.full((_L,), -jnp.inf, jnp.float32)
    zeros_i = jnp.zeros((_L,), jnp.int32)
    ones_i = jnp.ones((_L,), jnp.int32)

    wid = lax.axis_index("s") * 2 + lax.axis_index("c")

    def do_row(rr, _carry):
        row = wid * _ROWS_PER_W + rr

        pltpu.sync_copy(x_hbm.at[row], s_v)

        def mx_body(g, acc):
            return jnp.maximum(acc, s_v[pl.ds(g * _L, _L)])
        m = _vmax(lax.fori_loop(0, _NV, mx_body, neg_inf_v, unroll=8))

        def exp_body(g, acc):
            e = jnp.exp(s_v[pl.ds(g * _L, _L)] - m)
            s_v[pl.ds(g * _L, _L)] = e
            return acc + e
        ssum = jnp.sum(lax.fori_loop(0, _NV, exp_body,
                                     jnp.zeros((_L,), jnp.float32), unroll=8))

        def div_body(h, _):
            m1vec = neg_inf_v
            for i in range(_L):
                g = h * _L + i
                sv = s_v[pl.ds(g * _L, _L)] / ssum
                s_v[pl.ds(g * _L, _L)] = sv
                m1vec = jnp.where(iota == i, _vmax(sv), m1vec)
            m1_v[pl.ds(h * _L, _L)] = m1vec
            return 0
        lax.fori_loop(0, _NV // _L, div_body, 0)

        def m2_body(q, _):
            m2vec = neg_inf_v
            for i in range(_L):
                h = q * _L + i
                m2vec = jnp.where(iota == i, _vmax(m1_v[pl.ds(h * _L, _L)]),
                                  m2vec)
            m2_v[pl.ds(q * _L, _L)] = m2vec
            return 0
        lax.fori_loop(0, _NV // _L // _L, m2_body, 0)

        m3 = neg_inf_v
        for q in range(_NV // _L // _L):
            m3 = jnp.where(iota == q, _vmax(m2_v[pl.ds(q * _L, _L)]), m3)

        tk = [s_v[pl.ds(b * _L, _L)] for b in range(4)]
        tv = [iota + b * _L for b in range(4)]
        for b in range(4):
            tk[b], tv[b] = plsc.sort_key_val(tk[b], tv[b])
        (e0k, e1k), (e0v, e1v) = _merge16(tk[0], tv[0], tk[1], tv[1])
        (f0k, f1k), (f0v, f1v) = _merge16(tk[2], tv[2], tk[3], tv[3])
        sks, svs = _merge32((e0k, e1k), (e0v, e1v), (f0k, f1k), (f0v, f1v))
        for b in range(4):
            tkey_v[pl.ds(b * _L, _L)] = sks[b]
            tperm_v[pl.ds(b * _L, _L)] = svs[b]

        def ext_body(j, carry):
            m3c, vals, idxs = carry
            mv = _vmax(m3c)
            q = _ffs(m3c == mv)
            v2 = m2_v[pl.ds(q * _L, _L)]
            h = q * _L + _ffs(v2 == mv)
            v1 = m1_v[pl.ds(h * _L, _L)]
            g = h * _L + _ffs(v1 == mv)
            ve = s_v[pl.ds(g * _L, _L)]
            e2 = _ffs(ve == mv)
            flat = g * _L + e2
            jhi = j // _L
            jlo = j - jhi * _L
            vals = tuple(
                jnp.where((jhi == b) & (iota == jlo), mv, vals[b])
                for b in range(4))
            idxs = tuple(
                jnp.where((jhi == b) & (iota == jlo), flat, idxs[b])
                for b in range(4))
            ve = jnp.where(iota == e2, _NEG, ve)
            s_v[pl.ds(g * _L, _L)] = ve
            v1 = jnp.where(iota == (g - (g // _L) * _L), _vmax(ve), v1)
            m1_v[pl.ds(h * _L, _L)] = v1
            v2 = jnp.where(iota == (h - q * _L), _vmax(v1), v2)
            m2_v[pl.ds(q * _L, _L)] = v2
            m3c = jnp.where(iota == q, _vmax(v2), m3c)
            return m3c, vals, idxs

        init_vals = tuple(jnp.zeros((_L,), jnp.float32) for _ in range(4))
        init_idxs = tuple(zeros_i for _ in range(4))
        _, vals, idxs = lax.fori_loop(0, _TOP_K, ext_body,
                                      (m3, init_vals, init_idxs))

        for b in range(4):
            plsc.store_scatter(s_v, [idxs[b]], vals[b])

        run = jnp.float32(0.0)
        t_cnt = jnp.int32(0)
        for b in range(4):
            cum = plsc.cumsum(vals[b]) + run
            t_cnt = t_cnt + jnp.max(
                plsc.all_reduce_population_count(cum <= _TOP_P))
            run = run + jnp.sum(vals[b])
        kk = jnp.minimum(t_cnt + 1, _TOP_K)

        def hz_body(c, _):
            hist_v[pl.ds(c * _L, _L)] = zeros_i
            return 0
        lax.fori_loop(0, _TOP_K + 1, hz_body, 0, unroll=4)

        def bs_body(g, _):
            y = s_v[pl.ds(g * _L, _L)]
            pos = zeros_i
            for w in (32, 16, 8, 4, 2, 1):
                probe = pos + (w - 1)
                tkey = plsc.load_gather(tkey_v, [probe])
                pos = jnp.where(tkey < y, pos + w, pos)
            tkey = plsc.load_gather(tkey_v, [pos])
            pos = jnp.where(tkey < y, pos + 1, pos)
            plsc.addupdate_scatter(hist_v, [pos * _L + iota], ones_i)
            return 0
        lax.fori_loop(0, _NV, bs_body, 0, unroll=8)

        tot = []
        for b in range(4):
            acc = zeros_i
            base = (iota + b * _L + 1) * _L
            for l in range(_L):
                acc = acc + plsc.load_gather(hist_v, [base + l])
            tot.append(acc)
        carry_sum = jnp.int32(0)
        gvec = [None] * 4
        for b in (3, 2, 1, 0):
            rc = lax.rev(plsc.cumsum(lax.rev(tot[b], (0,))), (0,))
            gvec[b] = rc + carry_sum
            carry_sum = carry_sum + jnp.sum(tot[b])
        for b in range(4):
            plsc.store_scatter(rank_v, [svs[b]], gvec[b])

        def eq_body(u, eqs):
            su = plsc.load_gather(s_v, [zeros_i + u])
            new = []
            for b in range(4):
                gi = iota + b * _L
                sv = s_v[pl.ds(b * _L, _L)]
                new.append(eqs[b] +
                           jnp.where((sv == su) & (gi > u), 1, 0))
            return tuple(new)
        eqs = lax.fori_loop(0, _TOP_K - 1, eq_body,
                            tuple(zeros_i for _ in range(4)))

        def fill_body(g, _):
            o_v[pl.ds(g * _L, _L)] = neg_inf_v
            return 0
        lax.fori_loop(0, _NV, fill_body, 0, unroll=8)

        for b in range(4):
            rank_b = rank_v[pl.ds(b * _L, _L)] + eqs[b]
            keep = (iota + b * _L) < kk
            plsc.store_scatter(o_v, [rank_b], vals[b], mask=keep)

        pltpu.sync_copy(o_v, o_hbm.at[row])
        return 0

    lax.fori_loop(0, _ROWS_PER_W, do_row, 0)


@jax.jit
def kernel(x):
    mesh = plsc.VectorSubcoreMesh(core_axis_name="c", subcore_axis_name="s", num_cores=2, num_subcores=16)
    f = pl.kernel(
        _sc_body,
        out_type=jax.ShapeDtypeStruct((_ROWS, _N), jnp.float32),
        mesh=mesh,
        scratch_types=[
            pltpu.VMEM((_N,), jnp.float32),
            pltpu.VMEM((_N,), jnp.float32),
            pltpu.VMEM((_NV,), jnp.float32),
            pltpu.VMEM((_NV // _L,), jnp.float32),
            pltpu.VMEM((_TOP_K,), jnp.float32),
            pltpu.VMEM((_TOP_K,), jnp.int32),
            pltpu.VMEM((_TOP_K,), jnp.int32),
            pltpu.VMEM(((_TOP_K + 1) * _L,), jnp.int32),
            pltpu.SemaphoreType.DMA,
        ],
        compiler_params=pltpu.CompilerParams(needs_layout_passes=False),
    )
    return f(x)


# speedup vs baseline: 5.3146x; 1.3519x over previous
import functools

import jax
import jax.numpy as jnp
from jax import lax
from jax.experimental import pallas as pl
from jax.experimental.pallas import tpu as pltpu
from jax.experimental.pallas import tpu_sc as plsc

_TOP_K = 64
_TOP_P = 0.6
_N = 32768
_ROWS = 128
_L = 16
_NV = _N // _L
_NW = 32
_ROWS_PER_W = _ROWS // _NW
_NEG = -1.0

_IOTA = None


def _vmax(v):
    return jnp.max(v)


def _ffs(mask):
    return jnp.max(plsc.all_reduce_ffs(mask))


def _cminmax(ak, av, bk, bv):
    c = ak <= bk
    lo_k = jnp.where(c, ak, bk)
    lo_v = jnp.where(c, av, bv)
    hi_k = jnp.where(c, bk, ak)
    hi_v = jnp.where(c, bv, av)
    return lo_k, lo_v, hi_k, hi_v


def _merge16(ak, av, bk, bv):
    rbk = lax.rev(bk, (0,))
    rbv = lax.rev(bv, (0,))
    lo_k, lo_v, hi_k, hi_v = _cminmax(ak, av, rbk, rbv)
    lo_k, lo_v = plsc.sort_key_val(lo_k, lo_v)
    hi_k, hi_v = plsc.sort_key_val(hi_k, hi_v)
    return (lo_k, hi_k), (lo_v, hi_v)


def _merge32(aks, avs, bks, bvs):
    rbk = (lax.rev(bks[1], (0,)), lax.rev(bks[0], (0,)))
    rbv = (lax.rev(bvs[1], (0,)), lax.rev(bvs[0], (0,)))
    l0k, l0v, h0k, h0v = _cminmax(aks[0], avs[0], rbk[0], rbv[0])
    l1k, l1v, h1k, h1v = _cminmax(aks[1], avs[1], rbk[1], rbv[1])
    p0k, p0v, p1k, p1v = _cminmax(l0k, l0v, l1k, l1v)
    q0k, q0v, q1k, q1v = _cminmax(h0k, h0v, h1k, h1v)
    p0k, p0v = plsc.sort_key_val(p0k, p0v)
    p1k, p1v = plsc.sort_key_val(p1k, p1v)
    q0k, q0v = plsc.sort_key_val(q0k, q0v)
    q1k, q1v = plsc.sort_key_val(q1k, q1v)
    return (p0k, p1k, q0k, q1k), (p0v, p1v, q0v, q1v)


def _sc_body(x_hbm, o_hbm, s_v, o_v, m1_v, m2_v, tkey_v, tperm_v, rank_v,
             hist_v, sem):
    iota = lax.broadcasted_iota(jnp.int32, (_L,), 0)
    neg_inf_v = jnp.full((_L,), -jnp.inf, jnp.float32)
    zeros_i = jnp.zeros((_L,), jnp.int32)
    ones_i = jnp.ones((_L,), jnp.int32)

    wid = lax.axis_index("s") * 2 + lax.axis_index("c")

    def fill_body(g, _):
        o_v[pl.ds(g * _L, _L)] = neg_inf_v
        return 0
    lax.fori_loop(0, _NV, fill_body, 0, unroll=8)

    def do_row(rr, _carry):
        row = wid * _ROWS_PER_W + rr

        pltpu.sync_copy(x_hbm.at[row], s_v)

        def mx_body(g, accs):
            a0, a1, a2, a3 = accs
            a0 = jnp.maximum(a0, s_v[pl.ds((4 * g) * _L, _L)])
            a1 = jnp.maximum(a1, s_v[pl.ds((4 * g + 1) * _L, _L)])
            a2 = jnp.maximum(a2, s_v[pl.ds((4 * g + 2) * _L, _L)])
            a3 = jnp.maximum(a3, s_v[pl.ds((4 * g + 3) * _L, _L)])
            return a0, a1, a2, a3
        accs = lax.fori_loop(0, _NV // 4, mx_body, (neg_inf_v,) * 4,
                             unroll=4)
        m = _vmax(jnp.maximum(jnp.maximum(accs[0], accs[1]),
                              jnp.maximum(accs[2], accs[3])))

        def exp_body(h, accs):
            a0, a1 = accs
            m1vec = neg_inf_v
            for i in range(_L):
                g = h * _L + i
                e = jnp.exp(s_v[pl.ds(g * _L, _L)] - m)
                s_v[pl.ds(g * _L, _L)] = e
                if i % 2 == 0:
                    a0 = a0 + e
                else:
                    a1 = a1 + e
                m1vec = jnp.where(iota == i, _vmax(e), m1vec)
            m1_v[pl.ds(h * _L, _L)] = m1vec
            return a0, a1
        z = jnp.zeros((_L,), jnp.float32)
        sa0, sa1 = lax.fori_loop(0, _NV // _L, exp_body, (z, z))
        ssum = jnp.sum(sa0 + sa1)

        def m2_body(q, _):
            m2vec = neg_inf_v
            for i in range(_L):
                h = q * _L + i
                m2vec = jnp.where(iota == i, _vmax(m1_v[pl.ds(h * _L, _L)]),
                                  m2vec)
            m2_v[pl.ds(q * _L, _L)] = m2vec
            return 0
        lax.fori_loop(0, _NV // _L // _L, m2_body, 0)

        m3 = neg_inf_v
        for q in range(_NV // _L // _L):
            m3 = jnp.where(iota == q, _vmax(m2_v[pl.ds(q * _L, _L)]), m3)

        tk = [s_v[pl.ds(b * _L, _L)] for b in range(4)]
        tv = [iota + b * _L for b in range(4)]
        for b in range(4):
            tk[b], tv[b] = plsc.sort_key_val(tk[b], tv[b])
        (e0k, e1k), (e0v, e1v) = _merge16(tk[0], tv[0], tk[1], tv[1])
        (f0k, f1k), (f0v, f1v) = _merge16(tk[2], tv[2], tk[3], tv[3])
        sks, svs = _merge32((e0k, e1k), (e0v, e1v), (f0k, f1k), (f0v, f1v))
        for b in range(4):
            tkey_v[pl.ds(b * _L, _L)] = sks[b]
            tperm_v[pl.ds(b * _L, _L)] = svs[b]

        def ext_body(j, carry):
            m3c, vals, idxs = carry
            mv = _vmax(m3c)
            q = _ffs(m3c == mv)
            v2 = m2_v[pl.ds(q * _L, _L)]
            h = q * _L + _ffs(v2 == mv)
            v1 = m1_v[pl.ds(h * _L, _L)]
            g = h * _L + _ffs(v1 == mv)
            ve = s_v[pl.ds(g * _L, _L)]
            e2 = _ffs(ve == mv)
            flat = g * _L + e2
            jhi = j // _L
            jlo = j - jhi * _L
            vals = tuple(
                jnp.where((jhi == b) & (iota == jlo), mv, vals[b])
                for b in range(4))
            idxs = tuple(
                jnp.where((jhi == b) & (iota == jlo), flat, idxs[b])
                for b in range(4))
            ve = jnp.where(iota == e2, _NEG, ve)
            s_v[pl.ds(g * _L, _L)] = ve
            v1 = jnp.where(iota == (g - (g // _L) * _L), _vmax(ve), v1)
            m1_v[pl.ds(h * _L, _L)] = v1
            v2 = jnp.where(iota == (h - q * _L), _vmax(v1), v2)
            m2_v[pl.ds(q * _L, _L)] = v2
            m3c = jnp.where(iota == q, _vmax(v2), m3c)
            return m3c, vals, idxs

        init_vals = tuple(jnp.zeros((_L,), jnp.float32) for _ in range(4))
        init_idxs = tuple(zeros_i for _ in range(4))
        _, vals, idxs = lax.fori_loop(0, _TOP_K, ext_body,
                                      (m3, init_vals, init_idxs))

        for b in range(4):
            plsc.store_scatter(s_v, [idxs[b]], vals[b])

        svals = tuple(vals[b] / ssum for b in range(4))

        run = jnp.float32(0.0)
        t_cnt = jnp.int32(0)
        for b in range(4):
            cum = plsc.cumsum(svals[b]) + run
            t_cnt = t_cnt + jnp.max(
                plsc.all_reduce_population_count(cum <= _TOP_P))
            run = run + jnp.sum(svals[b])
        kk = jnp.minimum(t_cnt + 1, _TOP_K)

        def hz_body(c, _):
            hist_v[pl.ds(c * _L, _L)] = zeros_i
            return 0
        lax.fori_loop(0, _TOP_K + 1, hz_body, 0, unroll=4)

        t15 = _vmax(sks[0])
        t31 = _vmax(sks[1])
        t47 = _vmax(sks[2])
        t63 = _vmax(sks[3])

        def bs_body(g, _):
            y = s_v[pl.ds(g * _L, _L)]
            c32 = t31 < y
            pos = jnp.where(c32, 32, 0)
            piv = jnp.where(c32, t47, t15)
            pos = jnp.where(piv < y, pos + 16, pos)
            for w in (8, 4, 2, 1):
                probe = pos + (w - 1)
                tkey = plsc.load_gather(tkey_v, [probe])
                pos = jnp.where(tkey < y, pos + w, pos)
            pos = jnp.where((pos == 63) & (t63 < y), 64, pos)
            plsc.addupdate_scatter(hist_v, [pos * _L + iota], ones_i)
            return 0
        lax.fori_loop(0, _NV, bs_body, 0, unroll=8)

        tot = []
        for b in range(4):
            acc = zeros_i
            base = (iota + b * _L + 1) * _L
            for l in range(_L):
                acc = acc + plsc.load_gather(hist_v, [base + l])
            tot.append(acc)
        carry_sum = jnp.int32(0)
        gvec = [None] * 4
        for b in (3, 2, 1, 0):
            rc = lax.rev(plsc.cumsum(lax.rev(tot[b], (0,))), (0,))
            gvec[b] = rc + carry_sum
            carry_sum = carry_sum + jnp.sum(tot[b])
        for b in range(4):
            plsc.store_scatter(rank_v, [svs[b]], gvec[b])

        def eq_body(u, eqs):
            su = plsc.load_gather(s_v, [zeros_i + u])
            new = []
            for b in range(4):
                gi = iota + b * _L
                sv = s_v[pl.ds(b * _L, _L)]
                new.append(eqs[b] +
                           jnp.where((sv == su) & (gi > u), 1, 0))
            return tuple(new)
        eqs = lax.fori_loop(0, _TOP_K - 1, eq_body,
                            tuple(zeros_i for _ in range(4)))

        ranks = []
        keeps = []
        for b in range(4):
            rank_b = rank_v[pl.ds(b * _L, _L)] + eqs[b]
            keep = (iota + b * _L) < kk
            ranks.append(rank_b)
            keeps.append(keep)
            plsc.store_scatter(o_v, [rank_b], svals[b], mask=keep)

        pltpu.sync_copy(o_v, o_hbm.at[row])

        for b in range(4):
            plsc.store_scatter(o_v, [ranks[b]], neg_inf_v, mask=keeps[b])
        return 0

    lax.fori_loop(0, _ROWS_PER_W, do_row, 0)


@jax.jit
def kernel(x):
    mesh = plsc.VectorSubcoreMesh(core_axis_name="c", subcore_axis_name="s", num_cores=2, num_subcores=16)
    f = pl.kernel(
        _sc_body,
        out_type=jax.ShapeDtypeStruct((_ROWS, _N), jnp.float32),
        mesh=mesh,
        scratch_types=[
            pltpu.VMEM((_N,), jnp.float32),
            pltpu.VMEM((_N,), jnp.float32),
            pltpu.VMEM((_NV,), jnp.float32),
            pltpu.VMEM((_NV // _L,), jnp.float32),
            pltpu.VMEM((_TOP_K,), jnp.float32),
            pltpu.VMEM((_TOP_K,), jnp.int32),
            pltpu.VMEM((_TOP_K,), jnp.int32),
            pltpu.VMEM(((_TOP_K + 1) * _L,), jnp.int32),
            pltpu.SemaphoreType.DMA,
        ],
        compiler_params=pltpu.CompilerParams(needs_layout_passes=False),
    )
    return f(x)


# speedup vs baseline: 5.3161x; 1.0003x over previous
import functools

import jax
import jax.numpy as jnp
from jax import lax
from jax.experimental import pallas as pl
from jax.experimental.pallas import tpu as pltpu
from jax.experimental.pallas import tpu_sc as plsc

_TOP_K = 64
_TOP_P = 0.6
_N = 32768
_ROWS = 128
_L = 16
_NV = _N // _L
_NW = 32
_ROWS_PER_W = _ROWS // _NW
_NEG = -1.0

_IOTA = None


def _vmax(v):
    return jnp.max(v)


def _ffs(mask):
    return jnp.max(plsc.all_reduce_ffs(mask))


def _cminmax(ak, av, bk, bv):
    c = ak <= bk
    lo_k = jnp.where(c, ak, bk)
    lo_v = jnp.where(c, av, bv)
    hi_k = jnp.where(c, bk, ak)
    hi_v = jnp.where(c, bv, av)
    return lo_k, lo_v, hi_k, hi_v


def _merge16(ak, av, bk, bv):
    rbk = lax.rev(bk, (0,))
    rbv = lax.rev(bv, (0,))
    lo_k, lo_v, hi_k, hi_v = _cminmax(ak, av, rbk, rbv)
    lo_k, lo_v = plsc.sort_key_val(lo_k, lo_v)
    hi_k, hi_v = plsc.sort_key_val(hi_k, hi_v)
    return (lo_k, hi_k), (lo_v, hi_v)


def _merge32(aks, avs, bks, bvs):
    rbk = (lax.rev(bks[1], (0,)), lax.rev(bks[0], (0,)))
    rbv = (lax.rev(bvs[1], (0,)), lax.rev(bvs[0], (0,)))
    l0k, l0v, h0k, h0v = _cminmax(aks[0], avs[0], rbk[0], rbv[0])
    l1k, l1v, h1k, h1v = _cminmax(aks[1], avs[1], rbk[1], rbv[1])
    p0k, p0v, p1k, p1v = _cminmax(l0k, l0v, l1k, l1v)
    q0k, q0v, q1k, q1v = _cminmax(h0k, h0v, h1k, h1v)
    p0k, p0v = plsc.sort_key_val(p0k, p0v)
    p1k, p1v = plsc.sort_key_val(p1k, p1v)
    q0k, q0v = plsc.sort_key_val(q0k, q0v)
    q1k, q1v = plsc.sort_key_val(q1k, q1v)
    return (p0k, p1k, q0k, q1k), (p0v, p1v, q0v, q1v)


def _sc_body(x_hbm, o_hbm, s_v, o_v, m1_v, m2_v, tkey_v, tperm_v, rank_v,
             hist_v, sem):
    iota = lax.broadcasted_iota(jnp.int32, (_L,), 0)
    neg_inf_v = jnp.full((_L,), -jnp.inf, jnp.float32)
    zeros_i = jnp.zeros((_L,), jnp.int32)
    ones_i = jnp.ones((_L,), jnp.int32)

    wid = lax.axis_index("s") * 2 + lax.axis_index("c")

    def fill_body(g, _):
        o_v[pl.ds(g * _L, _L)] = neg_inf_v
        return 0
    lax.fori_loop(0, _NV, fill_body, 0, unroll=8)

    def do_row(rr, _carry):
        row = wid * _ROWS_PER_W + rr

        pltpu.sync_copy(x_hbm.at[row], s_v)

        def mx_body(g, accs):
            a0, a1, a2, a3 = accs
            a0 = jnp.maximum(a0, s_v[pl.ds((4 * g) * _L, _L)])
            a1 = jnp.maximum(a1, s_v[pl.ds((4 * g + 1) * _L, _L)])
            a2 = jnp.maximum(a2, s_v[pl.ds((4 * g + 2) * _L, _L)])
            a3 = jnp.maximum(a3, s_v[pl.ds((4 * g + 3) * _L, _L)])
            return a0, a1, a2, a3
        with jax.named_scope("p1_max"):
            accs = lax.fori_loop(0, _NV // 4, mx_body, (neg_inf_v,) * 4,
                                 unroll=4)
        m = _vmax(jnp.maximum(jnp.maximum(accs[0], accs[1]),
                              jnp.maximum(accs[2], accs[3])))

        def exp_body(h, accs):
            a0, a1 = accs
            m1vec = neg_inf_v
            for i in range(_L):
                g = h * _L + i
                e = jnp.exp(s_v[pl.ds(g * _L, _L)] - m)
                s_v[pl.ds(g * _L, _L)] = e
                if i % 2 == 0:
                    a0 = a0 + e
                else:
                    a1 = a1 + e
                m1vec = jnp.where(iota == i, _vmax(e), m1vec)
            m1_v[pl.ds(h * _L, _L)] = m1vec
            return a0, a1
        z = jnp.zeros((_L,), jnp.float32)
        with jax.named_scope("p2_exp"):
            sa0, sa1 = lax.fori_loop(0, _NV // _L, exp_body, (z, z))
        ssum = jnp.sum(sa0 + sa1)

        def m2_body(q, _):
            m2vec = neg_inf_v
            for i in range(_L):
                h = q * _L + i
                m2vec = jnp.where(iota == i, _vmax(m1_v[pl.ds(h * _L, _L)]),
                                  m2vec)
            m2_v[pl.ds(q * _L, _L)] = m2vec
            return 0
        lax.fori_loop(0, _NV // _L // _L, m2_body, 0)

        m3 = neg_inf_v
        for q in range(_NV // _L // _L):
            m3 = jnp.where(iota == q, _vmax(m2_v[pl.ds(q * _L, _L)]), m3)

        tk = [s_v[pl.ds(b * _L, _L)] for b in range(4)]
        tv = [iota + b * _L for b in range(4)]
        for b in range(4):
            tk[b], tv[b] = plsc.sort_key_val(tk[b], tv[b])
        (e0k, e1k), (e0v, e1v) = _merge16(tk[0], tv[0], tk[1], tv[1])
        (f0k, f1k), (f0v, f1v) = _merge16(tk[2], tv[2], tk[3], tv[3])
        sks, svs = _merge32((e0k, e1k), (e0v, e1v), (f0k, f1k), (f0v, f1v))
        for b in range(4):
            tkey_v[pl.ds(b * _L, _L)] = sks[b]
            tperm_v[pl.ds(b * _L, _L)] = svs[b]

        def ext_body(j, carry):
            m3c, vals, idxs = carry
            mv = _vmax(m3c)
            q = _ffs(m3c == mv)
            v2 = m2_v[pl.ds(q * _L, _L)]
            h = q * _L + _ffs(v2 == mv)
            v1 = m1_v[pl.ds(h * _L, _L)]
            g = h * _L + _ffs(v1 == mv)
            ve = s_v[pl.ds(g * _L, _L)]
            e2 = _ffs(ve == mv)
            flat = g * _L + e2
            jhi = j // _L
            jlo = j - jhi * _L
            vals = tuple(
                jnp.where((jhi == b) & (iota == jlo), mv, vals[b])
                for b in range(4))
            idxs = tuple(
                jnp.where((jhi == b) & (iota == jlo), flat, idxs[b])
                for b in range(4))
            ve = jnp.where(iota == e2, _NEG, ve)
            s_v[pl.ds(g * _L, _L)] = ve
            v1 = jnp.where(iota == (g - (g // _L) * _L), _vmax(ve), v1)
            m1_v[pl.ds(h * _L, _L)] = v1
            v2 = jnp.where(iota == (h - q * _L), _vmax(v1), v2)
            m2_v[pl.ds(q * _L, _L)] = v2
            m3c = jnp.where(iota == q, _vmax(v2), m3c)
            return m3c, vals, idxs

        init_vals = tuple(jnp.zeros((_L,), jnp.float32) for _ in range(4))
        init_idxs = tuple(zeros_i for _ in range(4))
        with jax.named_scope("p3_extract"):
            _, vals, idxs = lax.fori_loop(0, _TOP_K, ext_body,
                                          (m3, init_vals, init_idxs))

        for b in range(4):
            plsc.store_scatter(s_v, [idxs[b]], vals[b])

        svals = tuple(vals[b] / ssum for b in range(4))

        run = jnp.float32(0.0)
        t_cnt = jnp.int32(0)
        for b in range(4):
            cum = plsc.cumsum(svals[b]) + run
            t_cnt = t_cnt + jnp.max(
                plsc.all_reduce_population_count(cum <= _TOP_P))
            run = run + jnp.sum(svals[b])
        kk = jnp.minimum(t_cnt + 1, _TOP_K)

        def hz_body(c, _):
            hist_v[pl.ds(c * _L, _L)] = zeros_i
            return 0
        lax.fori_loop(0, _TOP_K + 1, hz_body, 0, unroll=4)

        t15 = _vmax(sks[0])
        t31 = _vmax(sks[1])
        t47 = _vmax(sks[2])
        t63 = _vmax(sks[3])

        def bs_body(g, _):
            y = s_v[pl.ds(g * _L, _L)]
            c32 = t31 < y
            pos = jnp.where(c32, 32, 0)
            piv = jnp.where(c32, t47, t15)
            pos = jnp.where(piv < y, pos + 16, pos)
            for w in (8, 4, 2, 1):
                probe = pos + (w - 1)
                tkey = plsc.load_gather(tkey_v, [probe])
                pos = jnp.where(tkey < y, pos + w, pos)
            pos = jnp.where((pos == 63) & (t63 < y), 64, pos)
            plsc.addupdate_scatter(hist_v, [pos * _L + iota], ones_i)
            return 0
        with jax.named_scope("p4_bsearch"):
            lax.fori_loop(0, _NV, bs_body, 0, unroll=8)

        tot = []
        for b in range(4):
            acc = zeros_i
            base = (iota + b * _L + 1) * _L
            for l in range(_L):
                acc = acc + plsc.load_gather(hist_v, [base + l])
            tot.append(acc)
        carry_sum = jnp.int32(0)
        gvec = [None] * 4
        for b in (3, 2, 1, 0):
            rc = lax.rev(plsc.cumsum(lax.rev(tot[b], (0,))), (0,))
            gvec[b] = rc + carry_sum
            carry_sum = carry_sum + jnp.sum(tot[b])
        for b in range(4):
            plsc.store_scatter(rank_v, [svs[b]], gvec[b])

        def eq_body(u, eqs):
            su = plsc.load_gather(s_v, [zeros_i + u])
            new = []
            for b in range(4):
                gi = iota + b * _L
                sv = s_v[pl.ds(b * _L, _L)]
                new.append(eqs[b] +
                           jnp.where((sv == su) & (gi > u), 1, 0))
            return tuple(new)
        eqs = lax.fori_loop(0, _TOP_K - 1, eq_body,
                            tuple(zeros_i for _ in range(4)))

        ranks = []
        keeps = []
        for b in range(4):
            rank_b = rank_v[pl.ds(b * _L, _L)] + eqs[b]
            keep = (iota + b * _L) < kk
            ranks.append(rank_b)
            keeps.append(keep)
            plsc.store_scatter(o_v, [rank_b], svals[b], mask=keep)

        with jax.named_scope("p5_dma_out"):
            pltpu.sync_copy(o_v, o_hbm.at[row])

        for b in range(4):
            plsc.store_scatter(o_v, [ranks[b]], neg_inf_v, mask=keeps[b])
        return 0

    lax.fori_loop(0, _ROWS_PER_W, do_row, 0)


@jax.jit
def kernel(x):
    mesh = plsc.VectorSubcoreMesh(core_axis_name="c", subcore_axis_name="s", num_cores=2, num_subcores=16)
    f = pl.kernel(
        _sc_body,
        out_type=jax.ShapeDtypeStruct((_ROWS, _N), jnp.float32),
        mesh=mesh,
        scratch_types=[
            pltpu.VMEM((_N,), jnp.float32),
            pltpu.VMEM((_N,), jnp.float32),
            pltpu.VMEM((_NV,), jnp.float32),
            pltpu.VMEM((_NV // _L,), jnp.float32),
            pltpu.VMEM((_TOP_K,), jnp.float32),
            pltpu.VMEM((_TOP_K,), jnp.int32),
            pltpu.VMEM((_TOP_K,), jnp.int32),
            pltpu.VMEM(((_TOP_K + 1) * _L,), jnp.int32),
            pltpu.SemaphoreType.DMA,
        ],
        compiler_params=pltpu.CompilerParams(needs_layout_passes=False),
    )
    return f(x)


# speedup vs baseline: 13.1799x; 2.4792x over previous
import functools

import jax
import jax.numpy as jnp
from jax import lax
from jax.experimental import pallas as pl
from jax.experimental.pallas import tpu as pltpu
from jax.experimental.pallas import tpu_sc as plsc

_TOP_K = 64
_TOP_P = 0.6
_N = 32768
_ROWS = 128
_L = 16
_NV = _N // _L
_NW = 32
_ROWS_PER_W = _ROWS // _NW
_NEG = -1.0

_IOTA = None


def _vmax(v):
    return jnp.max(v)


def _ffs(mask):
    return jnp.max(plsc.all_reduce_ffs(mask))


def _cminmax(ak, av, bk, bv):
    c = ak <= bk
    lo_k = jnp.where(c, ak, bk)
    lo_v = jnp.where(c, av, bv)
    hi_k = jnp.where(c, bk, ak)
    hi_v = jnp.where(c, bv, av)
    return lo_k, lo_v, hi_k, hi_v


def _merge16(ak, av, bk, bv):
    rbk = lax.rev(bk, (0,))
    rbv = lax.rev(bv, (0,))
    lo_k, lo_v, hi_k, hi_v = _cminmax(ak, av, rbk, rbv)
    lo_k, lo_v = plsc.sort_key_val(lo_k, lo_v)
    hi_k, hi_v = plsc.sort_key_val(hi_k, hi_v)
    return (lo_k, hi_k), (lo_v, hi_v)


def _merge32(aks, avs, bks, bvs):
    rbk = (lax.rev(bks[1], (0,)), lax.rev(bks[0], (0,)))
    rbv = (lax.rev(bvs[1], (0,)), lax.rev(bvs[0], (0,)))
    l0k, l0v, h0k, h0v = _cminmax(aks[0], avs[0], rbk[0], rbv[0])
    l1k, l1v, h1k, h1v = _cminmax(aks[1], avs[1], rbk[1], rbv[1])
    p0k, p0v, p1k, p1v = _cminmax(l0k, l0v, l1k, l1v)
    q0k, q0v, q1k, q1v = _cminmax(h0k, h0v, h1k, h1v)
    p0k, p0v = plsc.sort_key_val(p0k, p0v)
    p1k, p1v = plsc.sort_key_val(p1k, p1v)
    q0k, q0v = plsc.sort_key_val(q0k, q0v)
    q1k, q1v = plsc.sort_key_val(q1k, q1v)
    return (p0k, p1k, q0k, q1k), (p0v, p1v, q0v, q1v)


def _sc_body(x_hbm, o_hbm, s_v, o_v, m1_v, m2_v, tkey_v, tperm_v, rank_v,
             hist_v, sem):
    iota = lax.broadcasted_iota(jnp.int32, (_L,), 0)
    neg_inf_v = jnp.full((_L,), -jnp.inf, jnp.float32)
    zeros_i = jnp.zeros((_L,), jnp.int32)
    ones_i = jnp.ones((_L,), jnp.int32)

    wid = lax.axis_index("s") * 2 + lax.axis_index("c")

    def fill_body(g, _):
        o_v[pl.ds(g * _L, _L)] = neg_inf_v
        return 0
    lax.fori_loop(0, _NV, fill_body, 0, unroll=8)

    def do_row(rr, _carry):
        row = wid * _ROWS_PER_W + rr

        pltpu.sync_copy(x_hbm.at[row], s_v)

        def mx_body(g, accs):
            a0, a1, a2, a3 = accs
            a0 = jnp.maximum(a0, s_v[pl.ds((4 * g) * _L, _L)])
            a1 = jnp.maximum(a1, s_v[pl.ds((4 * g + 1) * _L, _L)])
            a2 = jnp.maximum(a2, s_v[pl.ds((4 * g + 2) * _L, _L)])
            a3 = jnp.maximum(a3, s_v[pl.ds((4 * g + 3) * _L, _L)])
            return a0, a1, a2, a3
        with jax.named_scope("p1_max"):
            accs = lax.fori_loop(0, _NV // 4, mx_body, (neg_inf_v,) * 4,
                                 unroll=4)
        m = _vmax(jnp.maximum(jnp.maximum(accs[0], accs[1]),
                              jnp.maximum(accs[2], accs[3])))

        def exp_body(h, accs):
            a0, a1 = accs
            m1vec = neg_inf_v
            for i in range(_L):
                g = h * _L + i
                e = jnp.exp(s_v[pl.ds(g * _L, _L)] - m)
                s_v[pl.ds(g * _L, _L)] = e
                if i % 2 == 0:
                    a0 = a0 + e
                else:
                    a1 = a1 + e
                m1vec = jnp.where(iota == i, _vmax(e), m1vec)
            m1_v[pl.ds(h * _L, _L)] = m1vec
            return a0, a1
        z = jnp.zeros((_L,), jnp.float32)
        with jax.named_scope("p2_exp"):
            sa0, sa1 = lax.fori_loop(0, _NV // _L, exp_body, (z, z))
        ssum = jnp.sum(sa0 + sa1)

        def m2_body(q, _):
            m2vec = neg_inf_v
            for i in range(_L):
                h = q * _L + i
                m2vec = jnp.where(iota == i, _vmax(m1_v[pl.ds(h * _L, _L)]),
                                  m2vec)
            m2_v[pl.ds(q * _L, _L)] = m2vec
            return 0
        lax.fori_loop(0, _NV // _L // _L, m2_body, 0)

        m3 = neg_inf_v
        for q in range(_NV // _L // _L):
            m3 = jnp.where(iota == q, _vmax(m2_v[pl.ds(q * _L, _L)]), m3)

        tk = [s_v[pl.ds(b * _L, _L)] for b in range(4)]
        tv = [iota + b * _L for b in range(4)]
        for b in range(4):
            tk[b], tv[b] = plsc.sort_key_val(tk[b], tv[b])
        (e0k, e1k), (e0v, e1v) = _merge16(tk[0], tv[0], tk[1], tv[1])
        (f0k, f1k), (f0v, f1v) = _merge16(tk[2], tv[2], tk[3], tv[3])
        sks, svs = _merge32((e0k, e1k), (e0v, e1v), (f0k, f1k), (f0v, f1v))
        for b in range(4):
            tkey_v[pl.ds(b * _L, _L)] = sks[b]
            tperm_v[pl.ds(b * _L, _L)] = svs[b]

        def ext_body(j, carry):
            m3c, vals, idxs = carry
            mv = _vmax(m3c)
            q = _ffs(m3c == mv)
            v2 = m2_v[pl.ds(q * _L, _L)]
            h = q * _L + _ffs(v2 == mv)
            v1 = m1_v[pl.ds(h * _L, _L)]
            g = h * _L + _ffs(v1 == mv)
            ve = s_v[pl.ds(g * _L, _L)]
            e2 = _ffs(ve == mv)
            flat = g * _L + e2
            jhi = j // _L
            jlo = j - jhi * _L
            vals = tuple(
                jnp.where((jhi == b) & (iota == jlo), mv, vals[b])
                for b in range(4))
            idxs = tuple(
                jnp.where((jhi == b) & (iota == jlo), flat, idxs[b])
                for b in range(4))
            ve = jnp.where(iota == e2, _NEG, ve)
            s_v[pl.ds(g * _L, _L)] = ve
            v1 = jnp.where(iota == (g - (g // _L) * _L), _vmax(ve), v1)
            m1_v[pl.ds(h * _L, _L)] = v1
            v2 = jnp.where(iota == (h - q * _L), _vmax(v1), v2)
            m2_v[pl.ds(q * _L, _L)] = v2
            m3c = jnp.where(iota == q, _vmax(v2), m3c)
            return m3c, vals, idxs

        init_vals = tuple(jnp.zeros((_L,), jnp.float32) for _ in range(4))
        init_idxs = tuple(zeros_i for _ in range(4))
        with jax.named_scope("p3_extract"):
            _, vals, idxs = lax.fori_loop(0, _TOP_K, ext_body,
                                          (m3, init_vals, init_idxs))

        for b in range(4):
            plsc.store_scatter(s_v, [idxs[b]], vals[b])

        svals = tuple(vals[b] / ssum for b in range(4))

        run = jnp.float32(0.0)
        t_cnt = jnp.int32(0)
        for b in range(4):
            cum = plsc.cumsum(svals[b]) + run
            t_cnt = t_cnt + jnp.max(
                plsc.all_reduce_population_count(cum <= _TOP_P))
            run = run + jnp.sum(svals[b])
        kk = jnp.minimum(t_cnt + 1, _TOP_K)

        def hz_body(c, _):
            hist_v[pl.ds(c * _L, _L)] = zeros_i
            return 0
        lax.fori_loop(0, _TOP_K + 1, hz_body, 0, unroll=4)

        t15 = _vmax(sks[0])
        t31 = _vmax(sks[1])
        t47 = _vmax(sks[2])
        t63 = _vmax(sks[3])

        with jax.named_scope("p4_bsearch"):
            @plsc.parallel_loop(0, _NV, unroll=8)
            def _bs_loop(g):
                y = s_v[pl.ds(g * _L, _L)]
                c32 = t31 < y
                pos = jnp.where(c32, 32, 0)
                piv = jnp.where(c32, t47, t15)
                pos = jnp.where(piv < y, pos + 16, pos)
                for w in (8, 4, 2, 1):
                    probe = pos + (w - 1)
                    tkey = plsc.load_gather(tkey_v, [probe])
                    pos = jnp.where(tkey < y, pos + w, pos)
                pos = jnp.where((pos == 63) & (t63 < y), 64, pos)
                plsc.addupdate_scatter(hist_v, [pos * _L + iota], ones_i)

        tot = []
        for b in range(4):
            acc = zeros_i
            base = (iota + b * _L + 1) * _L
            for l in range(_L):
                acc = acc + plsc.load_gather(hist_v, [base + l])
            tot.append(acc)
        carry_sum = jnp.int32(0)
        gvec = [None] * 4
        for b in (3, 2, 1, 0):
            rc = lax.rev(plsc.cumsum(lax.rev(tot[b], (0,))), (0,))
            gvec[b] = rc + carry_sum
            carry_sum = carry_sum + jnp.sum(tot[b])
        for b in range(4):
            plsc.store_scatter(rank_v, [svs[b]], gvec[b])

        def eq_body(u, eqs):
            su = plsc.load_gather(s_v, [zeros_i + u])
            new = []
            for b in range(4):
                gi = iota + b * _L
                sv = s_v[pl.ds(b * _L, _L)]
                new.append(eqs[b] +
                           jnp.where((sv == su) & (gi > u), 1, 0))
            return tuple(new)
        eqs = lax.fori_loop(0, _TOP_K - 1, eq_body,
                            tuple(zeros_i for _ in range(4)))

        ranks = []
        keeps = []
        for b in range(4):
            rank_b = rank_v[pl.ds(b * _L, _L)] + eqs[b]
            keep = (iota + b * _L) < kk
            ranks.append(rank_b)
            keeps.append(keep)
            plsc.store_scatter(o_v, [rank_b], svals[b], mask=keep)

        with jax.named_scope("p5_dma_out"):
            pltpu.sync_copy(o_v, o_hbm.at[row])

        for b in range(4):
            plsc.store_scatter(o_v, [ranks[b]], neg_inf_v, mask=keeps[b])
        return 0

    lax.fori_loop(0, _ROWS_PER_W, do_row, 0)


@jax.jit
def kernel(x):
    mesh = plsc.VectorSubcoreMesh(core_axis_name="c", subcore_axis_name="s", num_cores=2, num_subcores=16)
    f = pl.kernel(
        _sc_body,
        out_type=jax.ShapeDtypeStruct((_ROWS, _N), jnp.float32),
        mesh=mesh,
        scratch_types=[
            pltpu.VMEM((_N,), jnp.float32),
            pltpu.VMEM((_N,), jnp.float32),
            pltpu.VMEM((_NV,), jnp.float32),
            pltpu.VMEM((_NV // _L,), jnp.float32),
            pltpu.VMEM((_TOP_K,), jnp.float32),
            pltpu.VMEM((_TOP_K,), jnp.int32),
            pltpu.VMEM((_TOP_K,), jnp.int32),
            pltpu.VMEM(((_TOP_K + 1) * _L,), jnp.int32),
            pltpu.SemaphoreType.DMA,
        ],
        compiler_params=pltpu.CompilerParams(needs_layout_passes=False),
    )
    return f(x)


# speedup vs baseline: 14.6420x; 1.1109x over previous
import functools

import jax
import jax.numpy as jnp
from jax import lax
from jax.experimental import pallas as pl
from jax.experimental.pallas import tpu as pltpu
from jax.experimental.pallas import tpu_sc as plsc

_TOP_K = 64
_TOP_P = 0.6
_N = 32768
_ROWS = 128
_L = 16
_NV = _N // _L
_NW = 32
_ROWS_PER_W = _ROWS // _NW
_NEG = -1.0

_IOTA = None


def _vmax(v):
    return jnp.max(v)


def _ffs(mask):
    return jnp.max(plsc.all_reduce_ffs(mask))


def _cminmax(ak, av, bk, bv):
    c = ak <= bk
    lo_k = jnp.where(c, ak, bk)
    lo_v = jnp.where(c, av, bv)
    hi_k = jnp.where(c, bk, ak)
    hi_v = jnp.where(c, bv, av)
    return lo_k, lo_v, hi_k, hi_v


def _merge16(ak, av, bk, bv):
    rbk = lax.rev(bk, (0,))
    rbv = lax.rev(bv, (0,))
    lo_k, lo_v, hi_k, hi_v = _cminmax(ak, av, rbk, rbv)
    lo_k, lo_v = plsc.sort_key_val(lo_k, lo_v)
    hi_k, hi_v = plsc.sort_key_val(hi_k, hi_v)
    return (lo_k, hi_k), (lo_v, hi_v)


def _merge32(aks, avs, bks, bvs):
    rbk = (lax.rev(bks[1], (0,)), lax.rev(bks[0], (0,)))
    rbv = (lax.rev(bvs[1], (0,)), lax.rev(bvs[0], (0,)))
    l0k, l0v, h0k, h0v = _cminmax(aks[0], avs[0], rbk[0], rbv[0])
    l1k, l1v, h1k, h1v = _cminmax(aks[1], avs[1], rbk[1], rbv[1])
    p0k, p0v, p1k, p1v = _cminmax(l0k, l0v, l1k, l1v)
    q0k, q0v, q1k, q1v = _cminmax(h0k, h0v, h1k, h1v)
    p0k, p0v = plsc.sort_key_val(p0k, p0v)
    p1k, p1v = plsc.sort_key_val(p1k, p1v)
    q0k, q0v = plsc.sort_key_val(q0k, q0v)
    q1k, q1v = plsc.sort_key_val(q1k, q1v)
    return (p0k, p1k, q0k, q1k), (p0v, p1v, q0v, q1v)


def _sc_body(x_hbm, o_hbm, s_v, o_v, m1_v, m2_v, tkey_v, tperm_v, rank_v,
             hist_v, sem):
    iota = lax.broadcasted_iota(jnp.int32, (_L,), 0)
    neg_inf_v = jnp.full((_L,), -jnp.inf, jnp.float32)
    zeros_i = jnp.zeros((_L,), jnp.int32)
    ones_i = jnp.ones((_L,), jnp.int32)

    wid = lax.axis_index("s") * 2 + lax.axis_index("c")

    @plsc.parallel_loop(0, _NV, unroll=8)
    def _fill_loop(g):
        o_v[pl.ds(g * _L, _L)] = neg_inf_v

    def do_row(rr, _carry):
        row = wid * _ROWS_PER_W + rr

        pltpu.sync_copy(x_hbm.at[row], s_v)

        with jax.named_scope("p1_max"):
            @plsc.parallel_loop(0, _NV // 4, unroll=4,
                                carry=(neg_inf_v,) * 4)
            def mx_accs(g, accs):
                a0, a1, a2, a3 = accs
                a0 = jnp.maximum(a0, s_v[pl.ds((4 * g) * _L, _L)])
                a1 = jnp.maximum(a1, s_v[pl.ds((4 * g + 1) * _L, _L)])
                a2 = jnp.maximum(a2, s_v[pl.ds((4 * g + 2) * _L, _L)])
                a3 = jnp.maximum(a3, s_v[pl.ds((4 * g + 3) * _L, _L)])
                return a0, a1, a2, a3
        m = _vmax(jnp.maximum(jnp.maximum(mx_accs[0], mx_accs[1]),
                              jnp.maximum(mx_accs[2], mx_accs[3])))

        z = jnp.zeros((_L,), jnp.float32)
        with jax.named_scope("p2_exp"):
            @plsc.parallel_loop(0, _NV // _L, unroll=2, carry=(z, z))
            def exp_accs(h, accs):
                a0, a1 = accs
                m1vec = neg_inf_v
                for i in range(_L):
                    g = h * _L + i
                    e = jnp.exp(s_v[pl.ds(g * _L, _L)] - m)
                    s_v[pl.ds(g * _L, _L)] = e
                    if i % 2 == 0:
                        a0 = a0 + e
                    else:
                        a1 = a1 + e
                    m1vec = jnp.where(iota == i, _vmax(e), m1vec)
                m1_v[pl.ds(h * _L, _L)] = m1vec
                return a0, a1
        ssum = jnp.sum(exp_accs[0] + exp_accs[1])

        def m2_body(q, _):
            m2vec = neg_inf_v
            for i in range(_L):
                h = q * _L + i
                m2vec = jnp.where(iota == i, _vmax(m1_v[pl.ds(h * _L, _L)]),
                                  m2vec)
            m2_v[pl.ds(q * _L, _L)] = m2vec
            return 0
        lax.fori_loop(0, _NV // _L // _L, m2_body, 0)

        m3 = neg_inf_v
        for q in range(_NV // _L // _L):
            m3 = jnp.where(iota == q, _vmax(m2_v[pl.ds(q * _L, _L)]), m3)

        tk = [s_v[pl.ds(b * _L, _L)] for b in range(4)]
        tv = [iota + b * _L for b in range(4)]
        for b in range(4):
            tk[b], tv[b] = plsc.sort_key_val(tk[b], tv[b])
        (e0k, e1k), (e0v, e1v) = _merge16(tk[0], tv[0], tk[1], tv[1])
        (f0k, f1k), (f0v, f1v) = _merge16(tk[2], tv[2], tk[3], tv[3])
        sks, svs = _merge32((e0k, e1k), (e0v, e1v), (f0k, f1k), (f0v, f1v))
        for b in range(4):
            tkey_v[pl.ds(b * _L, _L)] = sks[b]
            tperm_v[pl.ds(b * _L, _L)] = svs[b]

        def ext_body(j, carry):
            m3c, vals, idxs = carry
            mv = _vmax(m3c)
            q = _ffs(m3c == mv)
            v2 = m2_v[pl.ds(q * _L, _L)]
            h = q * _L + _ffs(v2 == mv)
            v1 = m1_v[pl.ds(h * _L, _L)]
            g = h * _L + _ffs(v1 == mv)
            ve = s_v[pl.ds(g * _L, _L)]
            e2 = _ffs(ve == mv)
            flat = g * _L + e2
            jhi = j // _L
            jlo = j - jhi * _L
            vals = tuple(
                jnp.where((jhi == b) & (iota == jlo), mv, vals[b])
                for b in range(4))
            idxs = tuple(
                jnp.where((jhi == b) & (iota == jlo), flat, idxs[b])
                for b in range(4))
            ve = jnp.where(iota == e2, _NEG, ve)
            s_v[pl.ds(g * _L, _L)] = ve
            v1 = jnp.where(iota == (g - (g // _L) * _L), _vmax(ve), v1)
            m1_v[pl.ds(h * _L, _L)] = v1
            v2 = jnp.where(iota == (h - q * _L), _vmax(v1), v2)
            m2_v[pl.ds(q * _L, _L)] = v2
            m3c = jnp.where(iota == q, _vmax(v2), m3c)
            return m3c, vals, idxs

        init_vals = tuple(jnp.zeros((_L,), jnp.float32) for _ in range(4))
        init_idxs = tuple(zeros_i for _ in range(4))
        with jax.named_scope("p3_extract"):
            _, vals, idxs = lax.fori_loop(0, _TOP_K, ext_body,
                                          (m3, init_vals, init_idxs))

        for b in range(4):
            plsc.store_scatter(s_v, [idxs[b]], vals[b])

        svals = tuple(vals[b] / ssum for b in range(4))

        run = jnp.float32(0.0)
        t_cnt = jnp.int32(0)
        for b in range(4):
            cum = plsc.cumsum(svals[b]) + run
            t_cnt = t_cnt + jnp.max(
                plsc.all_reduce_population_count(cum <= _TOP_P))
            run = run + jnp.sum(svals[b])
        kk = jnp.minimum(t_cnt + 1, _TOP_K)

        def hz_body(c, _):
            hist_v[pl.ds(c * _L, _L)] = zeros_i
            return 0
        lax.fori_loop(0, _TOP_K + 1, hz_body, 0, unroll=4)

        t15 = _vmax(sks[0])
        t31 = _vmax(sks[1])
        t47 = _vmax(sks[2])
        t63 = _vmax(sks[3])
        t7 = _vmax(jnp.where(iota < 8, sks[0], neg_inf_v))
        t23 = _vmax(jnp.where(iota < 8, sks[1], neg_inf_v))
        t39 = _vmax(jnp.where(iota < 8, sks[2], neg_inf_v))
        t55 = _vmax(jnp.where(iota < 8, sks[3], neg_inf_v))

        with jax.named_scope("p4_bsearch"):
            @plsc.parallel_loop(0, _NV, unroll=16)
            def _bs_loop(g):
                y = s_v[pl.ds(g * _L, _L)]
                c32 = t31 < y
                pos = jnp.where(c32, 32, 0)
                piv = jnp.where(c32, t47, t15)
                c16 = piv < y
                pos = jnp.where(c16, pos + 16, pos)
                piv8 = jnp.where(c32, jnp.where(c16, t55, t39),
                                 jnp.where(c16, t23, t7))
                pos = jnp.where(piv8 < y, pos + 8, pos)
                for w in (4, 2, 1):
                    probe = pos + (w - 1)
                    tkey = plsc.load_gather(tkey_v, [probe])
                    pos = jnp.where(tkey < y, pos + w, pos)
                pos = jnp.where((pos == 63) & (t63 < y), 64, pos)
                plsc.addupdate_scatter(hist_v, [pos * _L + iota], ones_i)

        tot = []
        for b in range(4):
            acc = zeros_i
            base = (iota + b * _L + 1) * _L
            for l in range(_L):
                acc = acc + plsc.load_gather(hist_v, [base + l])
            tot.append(acc)
        carry_sum = jnp.int32(0)
        gvec = [None] * 4
        for b in (3, 2, 1, 0):
            rc = lax.rev(plsc.cumsum(lax.rev(tot[b], (0,))), (0,))
            gvec[b] = rc + carry_sum
            carry_sum = carry_sum + jnp.sum(tot[b])
        for b in range(4):
            plsc.store_scatter(rank_v, [svs[b]], gvec[b])

        def eq_body(u, eqs):
            su = plsc.load_gather(s_v, [zeros_i + u])
            new = []
            for b in range(4):
                gi = iota + b * _L
                sv = s_v[pl.ds(b * _L, _L)]
                new.append(eqs[b] +
                           jnp.where((sv == su) & (gi > u), 1, 0))
            return tuple(new)
        eqs = lax.fori_loop(0, _TOP_K - 1, eq_body,
                            tuple(zeros_i for _ in range(4)))

        ranks = []
        keeps = []
        for b in range(4):
            rank_b = rank_v[pl.ds(b * _L, _L)] + eqs[b]
            keep = (iota + b * _L) < kk
            ranks.append(rank_b)
            keeps.append(keep)
            plsc.store_scatter(o_v, [rank_b], svals[b], mask=keep)

        with jax.named_scope("p5_dma_out"):
            pltpu.sync_copy(o_v, o_hbm.at[row])

        for b in range(4):
            plsc.store_scatter(o_v, [ranks[b]], neg_inf_v, mask=keeps[b])
        return 0

    lax.fori_loop(0, _ROWS_PER_W, do_row, 0)


@jax.jit
def kernel(x):
    mesh = plsc.VectorSubcoreMesh(core_axis_name="c", subcore_axis_name="s", num_cores=2, num_subcores=16)
    f = pl.kernel(
        _sc_body,
        out_type=jax.ShapeDtypeStruct((_ROWS, _N), jnp.float32),
        mesh=mesh,
        scratch_types=[
            pltpu.VMEM((_N,), jnp.float32),
            pltpu.VMEM((_N,), jnp.float32),
            pltpu.VMEM((_NV,), jnp.float32),
            pltpu.VMEM((_NV // _L,), jnp.float32),
            pltpu.VMEM((_TOP_K,), jnp.float32),
            pltpu.VMEM((_TOP_K,), jnp.int32),
            pltpu.VMEM((_TOP_K,), jnp.int32),
            pltpu.VMEM(((_TOP_K + 1) * _L,), jnp.int32),
            pltpu.SemaphoreType.DMA,
        ],
        compiler_params=pltpu.CompilerParams(needs_layout_passes=False),
    )
    return f(x)


# speedup vs baseline: 16.1802x; 1.1051x over previous
import functools

import jax
import jax.numpy as jnp
from jax import lax
from jax.experimental import pallas as pl
from jax.experimental.pallas import tpu as pltpu
from jax.experimental.pallas import tpu_sc as plsc

_TOP_K = 64
_TOP_P = 0.6
_N = 32768
_ROWS = 128
_L = 16
_NV = _N // _L
_NW = 32
_ROWS_PER_W = _ROWS // _NW
_NEG = -1.0

_IOTA = None


def _vmax(v):
    return jnp.max(v)


def _ffs(mask):
    return jnp.max(plsc.all_reduce_ffs(mask))


def _cminmax(ak, av, bk, bv):
    c = ak <= bk
    lo_k = jnp.where(c, ak, bk)
    lo_v = jnp.where(c, av, bv)
    hi_k = jnp.where(c, bk, ak)
    hi_v = jnp.where(c, bv, av)
    return lo_k, lo_v, hi_k, hi_v


def _merge16(ak, av, bk, bv):
    rbk = lax.rev(bk, (0,))
    rbv = lax.rev(bv, (0,))
    lo_k, lo_v, hi_k, hi_v = _cminmax(ak, av, rbk, rbv)
    lo_k, lo_v = plsc.sort_key_val(lo_k, lo_v)
    hi_k, hi_v = plsc.sort_key_val(hi_k, hi_v)
    return (lo_k, hi_k), (lo_v, hi_v)


def _merge32(aks, avs, bks, bvs):
    rbk = (lax.rev(bks[1], (0,)), lax.rev(bks[0], (0,)))
    rbv = (lax.rev(bvs[1], (0,)), lax.rev(bvs[0], (0,)))
    l0k, l0v, h0k, h0v = _cminmax(aks[0], avs[0], rbk[0], rbv[0])
    l1k, l1v, h1k, h1v = _cminmax(aks[1], avs[1], rbk[1], rbv[1])
    p0k, p0v, p1k, p1v = _cminmax(l0k, l0v, l1k, l1v)
    q0k, q0v, q1k, q1v = _cminmax(h0k, h0v, h1k, h1v)
    p0k, p0v = plsc.sort_key_val(p0k, p0v)
    p1k, p1v = plsc.sort_key_val(p1k, p1v)
    q0k, q0v = plsc.sort_key_val(q0k, q0v)
    q1k, q1v = plsc.sort_key_val(q1k, q1v)
    return (p0k, p1k, q0k, q1k), (p0v, p1v, q0v, q1v)


def _sc_body(x_hbm, o_hbm, s_v, o_v, m1_v, m2_v, tkey_v, tperm_v, rank_v,
             hist_v, sem):
    iota = lax.broadcasted_iota(jnp.int32, (_L,), 0)
    neg_inf_v = jnp.full((_L,), -jnp.inf, jnp.float32)
    zeros_i = jnp.zeros((_L,), jnp.int32)
    ones_i = jnp.ones((_L,), jnp.int32)

    wid = lax.axis_index("s") * 2 + lax.axis_index("c")

    @plsc.parallel_loop(0, _NV, unroll=8)
    def _fill_loop(g):
        o_v[pl.ds(g * _L, _L)] = neg_inf_v

    def do_row(rr, _carry):
        row = wid * _ROWS_PER_W + rr

        pltpu.sync_copy(x_hbm.at[row], s_v)

        with jax.named_scope("p1_max"):
            @plsc.parallel_loop(0, _NV // 4, unroll=4,
                                carry=(neg_inf_v,) * 4)
            def mx_accs(g, accs):
                a0, a1, a2, a3 = accs
                a0 = jnp.maximum(a0, s_v[pl.ds((4 * g) * _L, _L)])
                a1 = jnp.maximum(a1, s_v[pl.ds((4 * g + 1) * _L, _L)])
                a2 = jnp.maximum(a2, s_v[pl.ds((4 * g + 2) * _L, _L)])
                a3 = jnp.maximum(a3, s_v[pl.ds((4 * g + 3) * _L, _L)])
                return a0, a1, a2, a3
        m = _vmax(jnp.maximum(jnp.maximum(mx_accs[0], mx_accs[1]),
                              jnp.maximum(mx_accs[2], mx_accs[3])))

        z = jnp.zeros((_L,), jnp.float32)
        with jax.named_scope("p2_exp"):
            @plsc.parallel_loop(0, _NV // _L, unroll=2, carry=(z, z))
            def exp_accs(h, accs):
                a0, a1 = accs
                m1vec = neg_inf_v
                for i in range(_L):
                    g = h * _L + i
                    e = jnp.exp(s_v[pl.ds(g * _L, _L)] - m)
                    s_v[pl.ds(g * _L, _L)] = e
                    if i % 2 == 0:
                        a0 = a0 + e
                    else:
                        a1 = a1 + e
                    m1vec = jnp.where(iota == i, _vmax(e), m1vec)
                m1_v[pl.ds(h * _L, _L)] = m1vec
                return a0, a1
        ssum = jnp.sum(exp_accs[0] + exp_accs[1])

        def m2_body(q, _):
            m2vec = neg_inf_v
            for i in range(_L):
                h = q * _L + i
                m2vec = jnp.where(iota == i, _vmax(m1_v[pl.ds(h * _L, _L)]),
                                  m2vec)
            m2_v[pl.ds(q * _L, _L)] = m2vec
            return 0
        lax.fori_loop(0, _NV // _L // _L, m2_body, 0)

        m3 = neg_inf_v
        for q in range(_NV // _L // _L):
            m3 = jnp.where(iota == q, _vmax(m2_v[pl.ds(q * _L, _L)]), m3)

        tk = [s_v[pl.ds(b * _L, _L)] for b in range(4)]
        tv = [iota + b * _L for b in range(4)]
        for b in range(4):
            tk[b], tv[b] = plsc.sort_key_val(tk[b], tv[b])
        (e0k, e1k), (e0v, e1v) = _merge16(tk[0], tv[0], tk[1], tv[1])
        (f0k, f1k), (f0v, f1v) = _merge16(tk[2], tv[2], tk[3], tv[3])
        sks, svs = _merge32((e0k, e1k), (e0v, e1v), (f0k, f1k), (f0v, f1v))
        for b in range(4):
            tkey_v[pl.ds(b * _L, _L)] = sks[b]
            tperm_v[pl.ds(b * _L, _L)] = svs[b]

        def ext_body(j, carry):
            m3c, vals, idxs = carry
            mv = _vmax(m3c)
            q = _ffs(m3c == mv)
            v2 = m2_v[pl.ds(q * _L, _L)]
            h = q * _L + _ffs(v2 == mv)
            v1 = m1_v[pl.ds(h * _L, _L)]
            g = h * _L + _ffs(v1 == mv)
            ve = s_v[pl.ds(g * _L, _L)]
            e2 = _ffs(ve == mv)
            flat = g * _L + e2
            jhi = j // _L
            jlo = j - jhi * _L
            vals = tuple(
                jnp.where((jhi == b) & (iota == jlo), mv, vals[b])
                for b in range(4))
            idxs = tuple(
                jnp.where((jhi == b) & (iota == jlo), flat, idxs[b])
                for b in range(4))
            ve = jnp.where(iota == e2, _NEG, ve)
            s_v[pl.ds(g * _L, _L)] = ve
            v1 = jnp.where(iota == (g - (g // _L) * _L), _vmax(ve), v1)
            m1_v[pl.ds(h * _L, _L)] = v1
            v2 = jnp.where(iota == (h - q * _L), _vmax(v1), v2)
            m2_v[pl.ds(q * _L, _L)] = v2
            m3c = jnp.where(iota == q, _vmax(v2), m3c)
            return m3c, vals, idxs

        init_vals = tuple(jnp.zeros((_L,), jnp.float32) for _ in range(4))
        init_idxs = tuple(zeros_i for _ in range(4))
        with jax.named_scope("p3_extract"):
            _, vals, idxs = lax.fori_loop(0, _TOP_K, ext_body,
                                          (m3, init_vals, init_idxs))

        for b in range(4):
            plsc.store_scatter(s_v, [idxs[b]], vals[b])

        svals = tuple(vals[b] / ssum for b in range(4))

        run = jnp.float32(0.0)
        t_cnt = jnp.int32(0)
        for b in range(4):
            cum = plsc.cumsum(svals[b]) + run
            t_cnt = t_cnt + jnp.max(
                plsc.all_reduce_population_count(cum <= _TOP_P))
            run = run + jnp.sum(svals[b])
        kk = jnp.minimum(t_cnt + 1, _TOP_K)

        @plsc.parallel_loop(0, _TOP_K + 1, unroll=4)
        def _hz_loop(c):
            hist_v[pl.ds(c * _L, _L)] = zeros_i

        t15 = _vmax(sks[0])
        t31 = _vmax(sks[1])
        t47 = _vmax(sks[2])
        t63 = _vmax(sks[3])
        t7 = _vmax(jnp.where(iota < 8, sks[0], neg_inf_v))
        t23 = _vmax(jnp.where(iota < 8, sks[1], neg_inf_v))
        t39 = _vmax(jnp.where(iota < 8, sks[2], neg_inf_v))
        t55 = _vmax(jnp.where(iota < 8, sks[3], neg_inf_v))

        with jax.named_scope("p4_bsearch"):
            @plsc.parallel_loop(0, _NV, unroll=32)
            def _bs_loop(g):
                y = s_v[pl.ds(g * _L, _L)]
                c32 = t31 < y
                pos = jnp.where(c32, 32, 0)
                piv = jnp.where(c32, t47, t15)
                c16 = piv < y
                pos = jnp.where(c16, pos + 16, pos)
                piv8 = jnp.where(c32, jnp.where(c16, t55, t39),
                                 jnp.where(c16, t23, t7))
                pos = jnp.where(piv8 < y, pos + 8, pos)
                for w in (4, 2, 1):
                    probe = pos + (w - 1)
                    tkey = plsc.load_gather(tkey_v, [probe])
                    pos = jnp.where(tkey < y, pos + w, pos)
                pos = jnp.where((pos == 63) & (t63 < y), 64, pos)
                plsc.addupdate_scatter(hist_v, [pos * _L + iota], ones_i)

        tot = []
        for b in range(4):
            acc = zeros_i
            base = (iota + b * _L + 1) * _L
            for l in range(_L):
                acc = acc + plsc.load_gather(hist_v, [base + l])
            tot.append(acc)
        carry_sum = jnp.int32(0)
        gvec = [None] * 4
        for b in (3, 2, 1, 0):
            rc = lax.rev(plsc.cumsum(lax.rev(tot[b], (0,))), (0,))
            gvec[b] = rc + carry_sum
            carry_sum = carry_sum + jnp.sum(tot[b])
        for b in range(4):
            plsc.store_scatter(rank_v, [svs[b]], gvec[b])

        def eq_body(u, eqs):
            su = plsc.load_gather(s_v, [zeros_i + u])
            new = []
            for b in range(4):
                gi = iota + b * _L
                sv = s_v[pl.ds(b * _L, _L)]
                new.append(eqs[b] +
                           jnp.where((sv == su) & (gi > u), 1, 0))
            return tuple(new)
        eqs = lax.fori_loop(0, _TOP_K - 1, eq_body,
                            tuple(zeros_i for _ in range(4)))

        ranks = []
        keeps = []
        for b in range(4):
            rank_b = rank_v[pl.ds(b * _L, _L)] + eqs[b]
            keep = (iota + b * _L) < kk
            ranks.append(rank_b)
            keeps.append(keep)
            plsc.store_scatter(o_v, [rank_b], svals[b], mask=keep)

        with jax.named_scope("p5_dma_out"):
            pltpu.sync_copy(o_v, o_hbm.at[row])

        for b in range(4):
            plsc.store_scatter(o_v, [ranks[b]], neg_inf_v, mask=keeps[b])
        return 0

    lax.fori_loop(0, _ROWS_PER_W, do_row, 0)


@jax.jit
def kernel(x):
    mesh = plsc.VectorSubcoreMesh(core_axis_name="c", subcore_axis_name="s", num_cores=2, num_subcores=16)
    f = pl.kernel(
        _sc_body,
        out_type=jax.ShapeDtypeStruct((_ROWS, _N), jnp.float32),
        mesh=mesh,
        scratch_types=[
            pltpu.VMEM((_N,), jnp.float32),
            pltpu.VMEM((_N,), jnp.float32),
            pltpu.VMEM((_NV,), jnp.float32),
            pltpu.VMEM((_NV // _L,), jnp.float32),
            pltpu.VMEM((_TOP_K,), jnp.float32),
            pltpu.VMEM((_TOP_K,), jnp.int32),
            pltpu.VMEM((_TOP_K,), jnp.int32),
            pltpu.VMEM(((_TOP_K + 1) * _L,), jnp.int32),
            pltpu.SemaphoreType.DMA,
        ],
        compiler_params=pltpu.CompilerParams(needs_layout_passes=False),
    )
    return f(x)


# speedup vs baseline: 16.2387x; 1.0036x over previous
import functools

import jax
import jax.numpy as jnp
from jax import lax
from jax.experimental import pallas as pl
from jax.experimental.pallas import tpu as pltpu
from jax.experimental.pallas import tpu_sc as plsc

_TOP_K = 64
_TOP_P = 0.6
_N = 32768
_ROWS = 128
_L = 16
_NV = _N // _L
_NW = 32
_ROWS_PER_W = _ROWS // _NW
_NEG = -1.0

_IOTA = None


def _vmax(v):
    return jnp.max(v)


def _ffs(mask):
    return jnp.max(plsc.all_reduce_ffs(mask))


def _cminmax(ak, av, bk, bv):
    c = ak <= bk
    lo_k = jnp.where(c, ak, bk)
    lo_v = jnp.where(c, av, bv)
    hi_k = jnp.where(c, bk, ak)
    hi_v = jnp.where(c, bv, av)
    return lo_k, lo_v, hi_k, hi_v


def _merge16(ak, av, bk, bv):
    rbk = lax.rev(bk, (0,))
    rbv = lax.rev(bv, (0,))
    lo_k, lo_v, hi_k, hi_v = _cminmax(ak, av, rbk, rbv)
    lo_k, lo_v = plsc.sort_key_val(lo_k, lo_v)
    hi_k, hi_v = plsc.sort_key_val(hi_k, hi_v)
    return (lo_k, hi_k), (lo_v, hi_v)


def _merge32(aks, avs, bks, bvs):
    rbk = (lax.rev(bks[1], (0,)), lax.rev(bks[0], (0,)))
    rbv = (lax.rev(bvs[1], (0,)), lax.rev(bvs[0], (0,)))
    l0k, l0v, h0k, h0v = _cminmax(aks[0], avs[0], rbk[0], rbv[0])
    l1k, l1v, h1k, h1v = _cminmax(aks[1], avs[1], rbk[1], rbv[1])
    p0k, p0v, p1k, p1v = _cminmax(l0k, l0v, l1k, l1v)
    q0k, q0v, q1k, q1v = _cminmax(h0k, h0v, h1k, h1v)
    p0k, p0v = plsc.sort_key_val(p0k, p0v)
    p1k, p1v = plsc.sort_key_val(p1k, p1v)
    q0k, q0v = plsc.sort_key_val(q0k, q0v)
    q1k, q1v = plsc.sort_key_val(q1k, q1v)
    return (p0k, p1k, q0k, q1k), (p0v, p1v, q0v, q1v)


def _sc_body(x_hbm, o_hbm, s_v, o_v, m1_v, m2_v, tkey_v, tperm_v, rank_v,
             hist_v, sem):
    iota = lax.broadcasted_iota(jnp.int32, (_L,), 0)
    neg_inf_v = jnp.full((_L,), -jnp.inf, jnp.float32)
    zeros_i = jnp.zeros((_L,), jnp.int32)
    ones_i = jnp.ones((_L,), jnp.int32)

    wid = lax.axis_index("s") * 2 + lax.axis_index("c")

    @plsc.parallel_loop(0, _NV, unroll=8)
    def _fill_loop(g):
        o_v[pl.ds(g * _L, _L)] = neg_inf_v

    def do_row(rr, _carry):
        row = wid * _ROWS_PER_W + rr

        pltpu.sync_copy(x_hbm.at[row], s_v)

        with jax.named_scope("p1_max"):
            @plsc.parallel_loop(0, _NV // 4, unroll=4,
                                carry=(neg_inf_v,) * 4)
            def mx_accs(g, accs):
                a0, a1, a2, a3 = accs
                a0 = jnp.maximum(a0, s_v[pl.ds((4 * g) * _L, _L)])
                a1 = jnp.maximum(a1, s_v[pl.ds((4 * g + 1) * _L, _L)])
                a2 = jnp.maximum(a2, s_v[pl.ds((4 * g + 2) * _L, _L)])
                a3 = jnp.maximum(a3, s_v[pl.ds((4 * g + 3) * _L, _L)])
                return a0, a1, a2, a3
        m = _vmax(jnp.maximum(jnp.maximum(mx_accs[0], mx_accs[1]),
                              jnp.maximum(mx_accs[2], mx_accs[3])))

        z = jnp.zeros((_L,), jnp.float32)
        with jax.named_scope("p2_exp"):
            @plsc.parallel_loop(0, _NV // _L, unroll=2, carry=(z, z))
            def exp_accs(h, accs):
                a0, a1 = accs
                m1vec = neg_inf_v
                for i in range(_L):
                    g = h * _L + i
                    e = jnp.exp(s_v[pl.ds(g * _L, _L)] - m)
                    s_v[pl.ds(g * _L, _L)] = e
                    if i % 2 == 0:
                        a0 = a0 + e
                    else:
                        a1 = a1 + e
                    m1vec = jnp.where(iota == i, _vmax(e), m1vec)
                m1_v[pl.ds(h * _L, _L)] = m1vec
                return a0, a1
        ssum = jnp.sum(exp_accs[0] + exp_accs[1])

        def m2_body(q, _):
            m2vec = neg_inf_v
            for i in range(_L):
                h = q * _L + i
                m2vec = jnp.where(iota == i, _vmax(m1_v[pl.ds(h * _L, _L)]),
                                  m2vec)
            m2_v[pl.ds(q * _L, _L)] = m2vec
            return 0
        lax.fori_loop(0, _NV // _L // _L, m2_body, 0)

        m3 = neg_inf_v
        for q in range(_NV // _L // _L):
            m3 = jnp.where(iota == q, _vmax(m2_v[pl.ds(q * _L, _L)]), m3)

        tk = [s_v[pl.ds(b * _L, _L)] for b in range(4)]
        tv = [iota + b * _L for b in range(4)]
        for b in range(4):
            tk[b], tv[b] = plsc.sort_key_val(tk[b], tv[b])
        (e0k, e1k), (e0v, e1v) = _merge16(tk[0], tv[0], tk[1], tv[1])
        (f0k, f1k), (f0v, f1v) = _merge16(tk[2], tv[2], tk[3], tv[3])
        sks, svs = _merge32((e0k, e1k), (e0v, e1v), (f0k, f1k), (f0v, f1v))
        for b in range(4):
            tkey_v[pl.ds(b * _L, _L)] = sks[b]
            tperm_v[pl.ds(b * _L, _L)] = svs[b]

        def ext_body(j, carry):
            m3c, vals, idxs = carry
            mv = _vmax(m3c)
            q = _ffs(m3c == mv)
            v2 = m2_v[pl.ds(q * _L, _L)]
            h = q * _L + _ffs(v2 == mv)
            v1 = m1_v[pl.ds(h * _L, _L)]
            g = h * _L + _ffs(v1 == mv)
            ve = s_v[pl.ds(g * _L, _L)]
            e2 = _ffs(ve == mv)
            flat = g * _L + e2
            jhi = j // _L
            jlo = j - jhi * _L
            vals = tuple(
                jnp.where((jhi == b) & (iota == jlo), mv, vals[b])
                for b in range(4))
            idxs = tuple(
                jnp.where((jhi == b) & (iota == jlo), flat, idxs[b])
                for b in range(4))
            ve = jnp.where(iota == e2, _NEG, ve)
            s_v[pl.ds(g * _L, _L)] = ve
            v1 = jnp.where(iota == (g - (g // _L) * _L), _vmax(ve), v1)
            m1_v[pl.ds(h * _L, _L)] = v1
            v2 = jnp.where(iota == (h - q * _L), _vmax(v1), v2)
            m2_v[pl.ds(q * _L, _L)] = v2
            m3c = jnp.where(iota == q, _vmax(v2), m3c)
            return m3c, vals, idxs

        init_vals = tuple(jnp.zeros((_L,), jnp.float32) for _ in range(4))
        init_idxs = tuple(zeros_i for _ in range(4))
        with jax.named_scope("p3_extract"):
            _, vals, idxs = lax.fori_loop(0, _TOP_K, ext_body,
                                          (m3, init_vals, init_idxs))

        for b in range(4):
            plsc.store_scatter(s_v, [idxs[b]], vals[b])

        svals = tuple(vals[b] / ssum for b in range(4))

        run = jnp.float32(0.0)
        t_cnt = jnp.int32(0)
        for b in range(4):
            cum = plsc.cumsum(svals[b]) + run
            t_cnt = t_cnt + jnp.max(
                plsc.all_reduce_population_count(cum <= _TOP_P))
            run = run + jnp.sum(svals[b])
        kk = jnp.minimum(t_cnt + 1, _TOP_K)

        @plsc.parallel_loop(0, _TOP_K + 1, unroll=4)
        def _hz_loop(c):
            hist_v[pl.ds(c * _L, _L)] = zeros_i

        t15 = _vmax(sks[0])
        t31 = _vmax(sks[1])
        t47 = _vmax(sks[2])
        t63 = _vmax(sks[3])
        t7 = _vmax(jnp.where(iota < 8, sks[0], neg_inf_v))
        t23 = _vmax(jnp.where(iota < 8, sks[1], neg_inf_v))
        t39 = _vmax(jnp.where(iota < 8, sks[2], neg_inf_v))
        t55 = _vmax(jnp.where(iota < 8, sks[3], neg_inf_v))

        with jax.named_scope("p4_bsearch"):
            @plsc.parallel_loop(0, _NV, unroll=32)
            def _bs_loop(g):
                y = s_v[pl.ds(g * _L, _L)]
                c32 = t31 < y
                pos = jnp.where(c32, 32, 0)
                piv = jnp.where(c32, t47, t15)
                c16 = piv < y
                pos = jnp.where(c16, pos + 16, pos)
                piv8 = jnp.where(c32, jnp.where(c16, t55, t39),
                                 jnp.where(c16, t23, t7))
                pos = jnp.where(piv8 < y, pos + 8, pos)
                for w in (4, 2, 1):
                    probe = pos + (w - 1)
                    tkey = plsc.load_gather(tkey_v, [probe])
                    pos = jnp.where(tkey < y, pos + w, pos)
                pos = jnp.where((pos == 63) & (t63 < y), 64, pos)
                plsc.addupdate_scatter(hist_v, [pos * _L + iota], ones_i)

        tot = []
        for b in range(4):
            acc = zeros_i
            base = (iota + b * _L + 1) * _L
            for l in range(_L):
                acc = acc + plsc.load_gather(hist_v, [base + l])
            tot.append(acc)
        carry_sum = jnp.int32(0)
        gvec = [None] * 4
        for b in (3, 2, 1, 0):
            rc = lax.rev(plsc.cumsum(lax.rev(tot[b], (0,))), (0,))
            gvec[b] = rc + carry_sum
            carry_sum = carry_sum + jnp.sum(tot[b])
        for b in range(4):
            plsc.store_scatter(rank_v, [svs[b]], gvec[b])

        def eq_body(u, eqs):
            su = plsc.load_gather(s_v, [zeros_i + u])
            new = []
            for b in range(4):
                gi = iota + b * _L
                sv = s_v[pl.ds(b * _L, _L)]
                new.append(eqs[b] +
                           jnp.where((sv == su) & (gi > u), 1, 0))
            return tuple(new)
        eqs = lax.fori_loop(0, _TOP_K - 1, eq_body,
                            tuple(zeros_i for _ in range(4)))

        prev_ranks, prev_keeps = _carry
        with jax.named_scope("p5_dma_drain"):
            pltpu.make_async_copy(o_v, o_hbm.at[row], sem).wait()
        for b in range(4):
            plsc.store_scatter(o_v, [prev_ranks[b]], neg_inf_v,
                               mask=prev_keeps[b] > 0)

        ranks = []
        keeps = []
        for b in range(4):
            rank_b = rank_v[pl.ds(b * _L, _L)] + eqs[b]
            keep = (iota + b * _L) < kk
            ranks.append(rank_b)
            keeps.append(keep.astype(jnp.int32))
            plsc.store_scatter(o_v, [rank_b], svals[b], mask=keep)

        pltpu.async_copy(o_v, o_hbm.at[row], sem)
        return tuple(ranks), tuple(keeps)

    first_row = wid * _ROWS_PER_W
    pltpu.async_copy(o_v, o_hbm.at[first_row], sem)
    init_carry = (tuple(zeros_i for _ in range(4)),
                  tuple(zeros_i for _ in range(4)))
    lax.fori_loop(0, _ROWS_PER_W, do_row, init_carry)
    pltpu.make_async_copy(o_v, o_hbm.at[first_row], sem).wait()


@jax.jit
def kernel(x):
    mesh = plsc.VectorSubcoreMesh(core_axis_name="c", subcore_axis_name="s", num_cores=2, num_subcores=16)
    f = pl.kernel(
        _sc_body,
        out_type=jax.ShapeDtypeStruct((_ROWS, _N), jnp.float32),
        mesh=mesh,
        scratch_types=[
            pltpu.VMEM((_N,), jnp.float32),
            pltpu.VMEM((_N,), jnp.float32),
            pltpu.VMEM((_NV,), jnp.float32),
            pltpu.VMEM((_NV // _L,), jnp.float32),
            pltpu.VMEM((_TOP_K,), jnp.float32),
            pltpu.VMEM((_TOP_K,), jnp.int32),
            pltpu.VMEM((_TOP_K,), jnp.int32),
            pltpu.VMEM(((_TOP_K + 1) * _L,), jnp.int32),
            pltpu.SemaphoreType.DMA,
        ],
        compiler_params=pltpu.CompilerParams(needs_layout_passes=False),
    )
    return f(x)


# speedup vs baseline: 19.4135x; 1.1955x over previous
import functools

import jax
import jax.numpy as jnp
from jax import lax
from jax.experimental import pallas as pl
from jax.experimental.pallas import tpu as pltpu
from jax.experimental.pallas import tpu_sc as plsc

_TOP_K = 64
_TOP_P = 0.6
_N = 32768
_ROWS = 128
_L = 16
_NV = _N // _L
_NW = 32
_ROWS_PER_W = _ROWS // _NW
_NEG = -1.0

_IOTA = None


def _vmax(v):
    return jnp.max(v)


def _ffs(mask):
    return jnp.max(plsc.all_reduce_ffs(mask))


def _cminmax(ak, av, bk, bv):
    c = ak <= bk
    lo_k = jnp.where(c, ak, bk)
    lo_v = jnp.where(c, av, bv)
    hi_k = jnp.where(c, bk, ak)
    hi_v = jnp.where(c, bv, av)
    return lo_k, lo_v, hi_k, hi_v


def _merge16(ak, av, bk, bv):
    rbk = lax.rev(bk, (0,))
    rbv = lax.rev(bv, (0,))
    lo_k, lo_v, hi_k, hi_v = _cminmax(ak, av, rbk, rbv)
    lo_k, lo_v = plsc.sort_key_val(lo_k, lo_v)
    hi_k, hi_v = plsc.sort_key_val(hi_k, hi_v)
    return (lo_k, hi_k), (lo_v, hi_v)


def _merge32(aks, avs, bks, bvs):
    rbk = (lax.rev(bks[1], (0,)), lax.rev(bks[0], (0,)))
    rbv = (lax.rev(bvs[1], (0,)), lax.rev(bvs[0], (0,)))
    l0k, l0v, h0k, h0v = _cminmax(aks[0], avs[0], rbk[0], rbv[0])
    l1k, l1v, h1k, h1v = _cminmax(aks[1], avs[1], rbk[1], rbv[1])
    p0k, p0v, p1k, p1v = _cminmax(l0k, l0v, l1k, l1v)
    q0k, q0v, q1k, q1v = _cminmax(h0k, h0v, h1k, h1v)
    p0k, p0v = plsc.sort_key_val(p0k, p0v)
    p1k, p1v = plsc.sort_key_val(p1k, p1v)
    q0k, q0v = plsc.sort_key_val(q0k, q0v)
    q1k, q1v = plsc.sort_key_val(q1k, q1v)
    return (p0k, p1k, q0k, q1k), (p0v, p1v, q0v, q1v)


def _sc_body(x_hbm, o_hbm, s_v, o_v, m1_v, m2_v, tkey_v, tperm_v, rank_v,
             hist_v, sem):
    iota = lax.broadcasted_iota(jnp.int32, (_L,), 0)
    neg_inf_v = jnp.full((_L,), -jnp.inf, jnp.float32)
    zeros_i = jnp.zeros((_L,), jnp.int32)
    ones_i = jnp.ones((_L,), jnp.int32)

    wid = lax.axis_index("s") * 2 + lax.axis_index("c")

    @plsc.parallel_loop(0, _NV, unroll=8)
    def _fill_loop(g):
        o_v[pl.ds(g * _L, _L)] = neg_inf_v

    def do_row(rr, _carry):
        row = wid * _ROWS_PER_W + rr

        pltpu.sync_copy(x_hbm.at[row], s_v)

        with jax.named_scope("p1_max"):
            @plsc.parallel_loop(0, _NV // 4, unroll=4,
                                carry=(neg_inf_v,) * 4)
            def mx_accs(g, accs):
                a0, a1, a2, a3 = accs
                a0 = jnp.maximum(a0, s_v[pl.ds((4 * g) * _L, _L)])
                a1 = jnp.maximum(a1, s_v[pl.ds((4 * g + 1) * _L, _L)])
                a2 = jnp.maximum(a2, s_v[pl.ds((4 * g + 2) * _L, _L)])
                a3 = jnp.maximum(a3, s_v[pl.ds((4 * g + 3) * _L, _L)])
                return a0, a1, a2, a3
        m = _vmax(jnp.maximum(jnp.maximum(mx_accs[0], mx_accs[1]),
                              jnp.maximum(mx_accs[2], mx_accs[3])))

        z = jnp.zeros((_L,), jnp.float32)
        with jax.named_scope("p2_exp"):
            @plsc.parallel_loop(0, _NV // _L, unroll=2, carry=(z, z))
            def exp_accs(h, accs):
                a0, a1 = accs
                m1vec = neg_inf_v
                for i in range(_L):
                    g = h * _L + i
                    e = jnp.exp(s_v[pl.ds(g * _L, _L)] - m)
                    s_v[pl.ds(g * _L, _L)] = e
                    if i % 2 == 0:
                        a0 = a0 + e
                    else:
                        a1 = a1 + e
                    m1vec = jnp.where(iota == i, _vmax(e), m1vec)
                m1_v[pl.ds(h * _L, _L)] = m1vec
                return a0, a1
        ssum = jnp.sum(exp_accs[0] + exp_accs[1])

        def m2_body(q, _):
            m2vec = neg_inf_v
            for i in range(_L):
                h = q * _L + i
                m2vec = jnp.where(iota == i, _vmax(m1_v[pl.ds(h * _L, _L)]),
                                  m2vec)
            m2_v[pl.ds(q * _L, _L)] = m2vec
            return 0
        lax.fori_loop(0, _NV // _L // _L, m2_body, 0)

        m3 = neg_inf_v
        for q in range(_NV // _L // _L):
            m3 = jnp.where(iota == q, _vmax(m2_v[pl.ds(q * _L, _L)]), m3)

        tk = [s_v[pl.ds(b * _L, _L)] for b in range(4)]
        tv = [iota + b * _L for b in range(4)]
        for b in range(4):
            tk[b], tv[b] = plsc.sort_key_val(tk[b], tv[b])
        (e0k, e1k), (e0v, e1v) = _merge16(tk[0], tv[0], tk[1], tv[1])
        (f0k, f1k), (f0v, f1v) = _merge16(tk[2], tv[2], tk[3], tv[3])
        sks, svs = _merge32((e0k, e1k), (e0v, e1v), (f0k, f1k), (f0v, f1v))
        for b in range(4):
            for l in range(_L):
                plsc.store_scatter(tkey_v, [(iota + b * _L) * _L + l], sks[b])
            tperm_v[pl.ds(b * _L, _L)] = svs[b]

        def ext_body(j, carry):
            m3c, vals, idxs = carry
            mv = _vmax(m3c)
            q = _ffs(m3c == mv)
            v2 = m2_v[pl.ds(q * _L, _L)]
            h = q * _L + _ffs(v2 == mv)
            v1 = m1_v[pl.ds(h * _L, _L)]
            g = h * _L + _ffs(v1 == mv)
            ve = s_v[pl.ds(g * _L, _L)]
            e2 = _ffs(ve == mv)
            flat = g * _L + e2
            jhi = j // _L
            jlo = j - jhi * _L
            vals = tuple(
                jnp.where((jhi == b) & (iota == jlo), mv, vals[b])
                for b in range(4))
            idxs = tuple(
                jnp.where((jhi == b) & (iota == jlo), flat, idxs[b])
                for b in range(4))
            ve = jnp.where(iota == e2, _NEG, ve)
            s_v[pl.ds(g * _L, _L)] = ve
            v1 = jnp.where(iota == (g - (g // _L) * _L), _vmax(ve), v1)
            m1_v[pl.ds(h * _L, _L)] = v1
            v2 = jnp.where(iota == (h - q * _L), _vmax(v1), v2)
            m2_v[pl.ds(q * _L, _L)] = v2
            m3c = jnp.where(iota == q, _vmax(v2), m3c)
            return m3c, vals, idxs

        init_vals = tuple(jnp.zeros((_L,), jnp.float32) for _ in range(4))
        init_idxs = tuple(zeros_i for _ in range(4))
        with jax.named_scope("p3_extract"):
            _, vals, idxs = lax.fori_loop(0, _TOP_K, ext_body,
                                          (m3, init_vals, init_idxs))

        for b in range(4):
            plsc.store_scatter(s_v, [idxs[b]], vals[b])

        svals = tuple(vals[b] / ssum for b in range(4))

        run = jnp.float32(0.0)
        t_cnt = jnp.int32(0)
        for b in range(4):
            cum = plsc.cumsum(svals[b]) + run
            t_cnt = t_cnt + jnp.max(
                plsc.all_reduce_population_count(cum <= _TOP_P))
            run = run + jnp.sum(svals[b])
        kk = jnp.minimum(t_cnt + 1, _TOP_K)

        @plsc.parallel_loop(0, _TOP_K + 1, unroll=4)
        def _hz_loop(c):
            hist_v[pl.ds(c * _L, _L)] = zeros_i

        t15 = _vmax(sks[0])
        t31 = _vmax(sks[1])
        t47 = _vmax(sks[2])
        t63 = _vmax(sks[3])
        t7 = _vmax(jnp.where(iota < 8, sks[0], neg_inf_v))
        t23 = _vmax(jnp.where(iota < 8, sks[1], neg_inf_v))
        t39 = _vmax(jnp.where(iota < 8, sks[2], neg_inf_v))
        t55 = _vmax(jnp.where(iota < 8, sks[3], neg_inf_v))

        with jax.named_scope("p4_bsearch"):
            @plsc.parallel_loop(0, _NV, unroll=32)
            def _bs_loop(g):
                y = s_v[pl.ds(g * _L, _L)]
                c32 = t31 < y
                pos = jnp.where(c32, 32, 0)
                piv = jnp.where(c32, t47, t15)
                c16 = piv < y
                pos = jnp.where(c16, pos + 16, pos)
                piv8 = jnp.where(c32, jnp.where(c16, t55, t39),
                                 jnp.where(c16, t23, t7))
                pos = jnp.where(piv8 < y, pos + 8, pos)
                for w in (4, 2, 1):
                    probe = pos + (w - 1)
                    tkey = plsc.load_gather(tkey_v, [probe * _L + iota])
                    pos = jnp.where(tkey < y, pos + w, pos)
                pos = jnp.where((pos == 63) & (t63 < y), 64, pos)
                plsc.addupdate_scatter(hist_v, [pos * _L + iota], ones_i)

        tot = []
        for b in range(4):
            acc = zeros_i
            base = (iota + b * _L + 1) * _L
            for l in range(_L):
                acc = acc + plsc.load_gather(hist_v, [base + l])
            tot.append(acc)
        carry_sum = jnp.int32(0)
        gvec = [None] * 4
        for b in (3, 2, 1, 0):
            rc = lax.rev(plsc.cumsum(lax.rev(tot[b], (0,))), (0,))
            gvec[b] = rc + carry_sum
            carry_sum = carry_sum + jnp.sum(tot[b])
        for b in range(4):
            plsc.store_scatter(rank_v, [svs[b]], gvec[b])

        def eq_body(u, eqs):
            su = plsc.load_gather(s_v, [zeros_i + u])
            new = []
            for b in range(4):
                gi = iota + b * _L
                sv = s_v[pl.ds(b * _L, _L)]
                new.append(eqs[b] +
                           jnp.where((sv == su) & (gi > u), 1, 0))
            return tuple(new)
        eqs = lax.fori_loop(0, _TOP_K - 1, eq_body,
                            tuple(zeros_i for _ in range(4)))

        prev_ranks, prev_keeps = _carry
        with jax.named_scope("p5_dma_drain"):
            pltpu.make_async_copy(o_v, o_hbm.at[row], sem).wait()
        for b in range(4):
            plsc.store_scatter(o_v, [prev_ranks[b]], neg_inf_v,
                               mask=prev_keeps[b] > 0)

        ranks = []
        keeps = []
        for b in range(4):
            rank_b = rank_v[pl.ds(b * _L, _L)] + eqs[b]
            keep = (iota + b * _L) < kk
            ranks.append(rank_b)
            keeps.append(keep.astype(jnp.int32))
            plsc.store_scatter(o_v, [rank_b], svals[b], mask=keep)

        pltpu.async_copy(o_v, o_hbm.at[row], sem)
        return tuple(ranks), tuple(keeps)

    first_row = wid * _ROWS_PER_W
    pltpu.async_copy(o_v, o_hbm.at[first_row], sem)
    init_carry = (tuple(zeros_i for _ in range(4)),
                  tuple(zeros_i for _ in range(4)))
    lax.fori_loop(0, _ROWS_PER_W, do_row, init_carry)
    pltpu.make_async_copy(o_v, o_hbm.at[first_row], sem).wait()


@jax.jit
def kernel(x):
    mesh = plsc.VectorSubcoreMesh(core_axis_name="c", subcore_axis_name="s", num_cores=2, num_subcores=16)
    f = pl.kernel(
        _sc_body,
        out_type=jax.ShapeDtypeStruct((_ROWS, _N), jnp.float32),
        mesh=mesh,
        scratch_types=[
            pltpu.VMEM((_N,), jnp.float32),
            pltpu.VMEM((_N,), jnp.float32),
            pltpu.VMEM((_NV,), jnp.float32),
            pltpu.VMEM((_NV // _L,), jnp.float32),
            pltpu.VMEM((_TOP_K * _L,), jnp.float32),
            pltpu.VMEM((_TOP_K,), jnp.int32),
            pltpu.VMEM((_TOP_K,), jnp.int32),
            pltpu.VMEM(((_TOP_K + 1) * _L,), jnp.int32),
            pltpu.SemaphoreType.DMA,
        ],
        compiler_params=pltpu.CompilerParams(needs_layout_passes=False),
    )
    return f(x)
